# Initial kernel scaffold; baseline (speedup 1.0000x reference)
import jax
import jax.numpy as jnp
from jax.experimental import pallas as pl


def kernel(x, edge_index, batch, W_lin, b_lin, bn1_g, bn1_b, gat1_W, gat1_as, gat1_ad, gat1_b, bn2_g, bn2_b, gat2_W, gat2_as, gat2_ad, gat2_b, k1, conv1_w, lin1_W, k2, conv2_w, lin2_W):
    raise NotImplementedError("write your pallas kernel here")


# speedup vs baseline: 34.2491x; 34.2491x over previous
import dataclasses
import functools

import jax
import jax.numpy as jnp
from jax import lax
from jax.experimental import pallas as pl
from jax.experimental.pallas import tpu as pltpu
from jax.experimental.pallas import tpu_sc as plsc

N = 10000
E = 320000
NPAD = 10016
EPAD = 327680
D = 48
HID = 32
NG = 16
NCLU = 10
NHEAD = 5
NC, NS = 2, 16
NW = NC * NS
EPW = EPAD // NW
CH = 256
CROWS = CH // 128
NCHUNK = EPW // CH
BLK = 1000
NBLK = N // BLK
EPS = 1e-15
_PREC = lax.Precision.HIGHEST


def _leaky(x, s):
    return jnp.where(x > 0, x, s * x)


def _dotT(a, b):
    return lax.dot_general(a, b, (((0,), (0,)), ((), ())),
                           preferred_element_type=jnp.float32,
                           precision=_PREC)


def _dot(a, b):
    return jnp.dot(a, b, preferred_element_type=jnp.float32, precision=_PREC)


def _linproj_body(x_ref, wl_ref, bl_ref, h0_ref):
    h0_ref[...] = _dot(x_ref[...], wl_ref[...]) + bl_ref[...]


def _prep_body(h_ref, g_ref, b_ref, w_ref, a_ref, hw_ref, svd_ref):
    h = h_ref[...]
    m = jnp.mean(h, axis=0)
    v = jnp.mean((h - m) ** 2, axis=0)
    hb = _leaky((h - m) / jnp.sqrt(v + 1e-5) * g_ref[...] + b_ref[...], 0.01)
    hw = _dot(hb, w_ref[...])
    svd = _dot(hw, a_ref[...])
    hw48 = jnp.concatenate(
        [hw, jnp.ones((N, 1), jnp.float32),
         jnp.zeros((N, D - HID - 1), jnp.float32)], axis=1)
    hw_ref[...] = jnp.concatenate(
        [hw48, jnp.zeros((NPAD - N, D), jnp.float32)], axis=0)
    svd_ref[...] = jnp.concatenate(
        [svd, jnp.zeros((NPAD - N, 2), jnp.float32)], axis=0)


def _combine_vals(accv, hwv, svdv, biasv, hprev):
    accs = accv[0] + accv[1]
    z = svdv[:, 0:1] + svdv[:, 1:2]
    ws = jnp.exp(_leaky(z, 0.2))
    num = accs[:, :HID] + ws * hwv[:, :HID]
    den = accs[:, HID:HID + 1] + ws
    return hprev + num / den + biasv


def _combine_body(h_ref, hw_ref, svd_ref, acc_ref, bias_ref, out_ref):
    out_ref[...] = _combine_vals(acc_ref[...], hw_ref[...], svd_ref[...],
                                 bias_ref[...], h_ref[...])


def _pool_body(h_ref, hw_ref, svd_ref, acc_ref, bias_ref, batch_ref,
               kf_ref, cw_ref, s_ref, cs_ref):
    h2 = _combine_vals(acc_ref[...], hw_ref[...], svd_ref[...],
                       bias_ref[...], h_ref[...])
    kf = kf_ref[...]
    hn2 = jnp.sum(h2 * h2, axis=1, keepdims=True)
    kn2 = jnp.sum(kf * kf, axis=1)
    d2 = hn2 + kn2[None, :] - 2.0 * lax.dot_general(
        h2, kf, (((1,), (1,)), ((), ())),
        preferred_element_type=jnp.float32, precision=_PREC)
    d2 = jnp.maximum(d2, 0.0)
    dist = 1.0 / (1.0 + d2)
    i50h = lax.broadcasted_iota(jnp.int32, (NHEAD * NCLU, NHEAD), 0)
    i5h = lax.broadcasted_iota(jnp.int32, (NHEAD * NCLU, NHEAD), 1)
    M5 = (i50h // NCLU == i5h).astype(jnp.float32)
    dsum = _dot(dist, M5)
    dfull = lax.dot_general(dsum, M5, (((1,), (1,)), ((), ())),
                            preferred_element_type=jnp.float32,
                            precision=_PREC)
    distn = dist / dfull
    cwexp = lax.dot_general(cw_ref[...], M5, (((1,), (1,)), ((), ())),
                            preferred_element_type=jnp.float32,
                            precision=_PREC)
    i50k = lax.broadcasted_iota(jnp.int32, (NHEAD * NCLU, NCLU), 0)
    i10k = lax.broadcasted_iota(jnp.int32, (NHEAD * NCLU, NCLU), 1)
    M10 = (i50k % NCLU == i10k).astype(jnp.float32)
    S = _dot(distn * cwexp, M10)
    mx = jnp.max(S, axis=1, keepdims=True)
    e = jnp.exp(S - mx)
    S = e / jnp.sum(e, axis=1, keepdims=True)
    s_ref[...] = S
    bt = batch_ref[...]
    i16 = lax.broadcasted_iota(jnp.int32, (1, NG), 1)
    G = (bt == i16).astype(jnp.float32)
    iR0 = lax.broadcasted_iota(jnp.int32, (NG, NG * NCLU), 0)
    iR1 = lax.broadcasted_iota(jnp.int32, (NG, NG * NCLU), 1)
    R = (iR1 // NCLU == iR0).astype(jnp.float32)
    iT0 = lax.broadcasted_iota(jnp.int32, (NCLU, NG * NCLU), 0)
    iT1 = lax.broadcasted_iota(jnp.int32, (NCLU, NG * NCLU), 1)
    T = (iT1 % NCLU == iT0).astype(jnp.float32)
    P = _dot(G, R) * _dot(S, T)
    ext = jnp.concatenate([h2, jnp.ones((h2.shape[0], 1), jnp.float32)],
                          axis=1)

    @pl.when(pl.program_id(0) == 0)
    def _():
        cs_ref[...] = jnp.zeros_like(cs_ref)

    cs_ref[...] += _dotT(P, ext)


def _head_body(s_ref, cs_ref, batch_ref, lin1_ref, lin2_ref,
               logp_ref, kl_ref):
    out_cs = cs_ref[...]
    out1 = out_cs[:, :HID]
    colsum = out_cs[:, HID:HID + 1]
    iR0 = lax.broadcasted_iota(jnp.int32, (NG, NG * NCLU), 0)
    iR1 = lax.broadcasted_iota(jnp.int32, (NG, NG * NCLU), 1)
    R = (iR1 // NCLU == iR0).astype(jnp.float32)
    iT0 = lax.broadcasted_iota(jnp.int32, (NCLU, NG * NCLU), 0)
    iT1 = lax.broadcasted_iota(jnp.int32, (NCLU, NG * NCLU), 1)
    T = (iT1 % NCLU == iT0).astype(jnp.float32)
    x1 = _leaky(_dot(out1, lin1_ref[...]), 0.01)
    x2 = _dot(_dot(R, x1), lin2_ref[...])
    mx2 = jnp.max(x2, axis=1, keepdims=True)
    lse = jnp.log(jnp.sum(jnp.exp(x2 - mx2), axis=1, keepdims=True)) + mx2
    logp_ref[...] = x2 - lse
    cs16 = lax.dot_general(R * jnp.transpose(colsum), T,
                           (((1,), (1,)), ((), ())),
                           preferred_element_type=jnp.float32,
                           precision=_PREC)
    bt = batch_ref[...]
    i16 = lax.broadcasted_iota(jnp.int32, (1, NG), 1)
    G = (bt == i16).astype(jnp.float32)
    cs_pn = _dot(G, cs16)
    S = s_ref[...]
    Pn = (S * S) / jnp.maximum(cs_pn, EPS)
    Pn = Pn / jnp.sum(Pn, axis=1, keepdims=True)
    Pc = jnp.maximum(Pn, EPS)
    Sc = jnp.maximum(S, EPS)
    kl = jnp.sum(Pc * (jnp.log(Pc) - jnp.log(Sc))) / NG
    kl_ref[...] = jnp.reshape(kl, (1, 1))


def _edge_body(hw_hbm, svd_hbm, src_hbm, dst_hbm, zero_hbm, out_hbm,
               svd_v, srcb, dstb, rows, wbuf, acc, sem):
    cid = lax.axis_index("c")
    sid = lax.axis_index("s")
    wid = sid * NC + cid

    @pl.when(sid == 0)
    def _():
        pltpu.sync_copy(zero_hbm, acc)

    pltpu.sync_copy(svd_hbm, svd_v)
    plsc.subcore_barrier()

    col0 = lax.iota(jnp.int32, 16) * 0
    col1 = col0 + 1
    row0 = wid * (EPW // 128)

    @pl.loop(0, NCHUNK)
    def _chunk(ci):
        r0 = row0 + ci * CROWS
        pltpu.sync_copy(src_hbm.at[pl.ds(r0, CROWS)], srcb)
        pltpu.sync_copy(dst_hbm.at[pl.ds(r0, CROWS)], dstb)
        descs = [
            pltpu.async_copy(hw_hbm.at[srcb.at[j]],
                             rows.at[pl.ds(j * 128, 128)], sem)
            for j in range(CROWS)
        ]
        for j in range(CROWS):
            @pl.loop(0, 128 // 16)
            def _grp(g, j=j):
                s16 = srcb[j, pl.ds(g * 16, 16)]
                d16 = dstb[j, pl.ds(g * 16, 16)]
                sv = plsc.load_gather(svd_v, [s16, col0])
                dv = plsc.load_gather(svd_v, [d16, col1])
                z = sv + dv
                z = jnp.where(z > 0, z, 0.2 * z)
                wbuf[pl.ds(j * 128 + g * 16, 16)] = jnp.exp(z)
        for dsc in descs:
            dsc.wait()

        @pl.loop(0, CH, unroll=4)
        def _mul(r):
            wv = plsc.load_gather(wbuf, [col0 + r])
            rows[r, pl.ds(0, 16)] = rows[r, pl.ds(0, 16)] * wv
            rows[r, pl.ds(16, 16)] = rows[r, pl.ds(16, 16)] * wv
            rows[r, pl.ds(32, 16)] = rows[r, pl.ds(32, 16)] * wv

        for j in range(CROWS):
            pltpu.sync_copy(rows.at[pl.ds(j * 128, 128)],
                            acc.at[dstb.at[j]], add=True)

    plsc.subcore_barrier()

    @pl.when(sid == 0)
    def _():
        pltpu.sync_copy(acc, out_hbm.at[cid])


def _edge_call(hw48, svd, src2d, dst2d, zeros, interpret=False):
    mesh = plsc.VectorSubcoreMesh(core_axis_name="c", subcore_axis_name="s",
                                  num_cores=NC, num_subcores=NS)
    cp = pltpu.CompilerParams(use_tc_tiling_on_sc=False)
    if "needs_layout_passes" in pltpu.CompilerParams.__dataclass_fields__:
        cp = dataclasses.replace(cp, needs_layout_passes=False)
    kern = pl.kernel(
        _edge_body,
        out_type=jax.ShapeDtypeStruct((NC, NPAD, D), jnp.float32),
        mesh=mesh,
        scratch_types=[
            pltpu.VMEM((NPAD, 2), jnp.float32),
            pltpu.VMEM((CROWS, 128), jnp.int32),
            pltpu.VMEM((CROWS, 128), jnp.int32),
            pltpu.VMEM((CH, D), jnp.float32),
            pltpu.VMEM((CH,), jnp.float32),
            pltpu.VMEM_SHARED((NPAD, D), jnp.float32),
            pltpu.SemaphoreType.DMA,
        ],
        interpret=interpret,
        compiler_params=cp,
    )
    return kern(hw48, svd, src2d, dst2d, zeros)


def _tc_call(body, out_shapes, args, grid=None, in_specs=None, out_specs=None,
             interpret=False):
    kwargs = {}
    if grid is not None:
        kwargs = dict(grid=grid, in_specs=in_specs, out_specs=out_specs)
    return pl.pallas_call(
        body,
        out_shape=out_shapes,
        interpret=interpret,
        **kwargs,
    )(*args)


def _full(shape):
    return pl.BlockSpec(shape, lambda i: tuple(0 for _ in shape))


def kernel(x, edge_index, batch, W_lin, b_lin, bn1_g, bn1_b, gat1_W, gat1_as,
           gat1_ad, gat1_b, bn2_g, bn2_b, gat2_W, gat2_as, gat2_ad, gat2_b,
           k1, conv1_w, lin1_W, k2, conv2_w, lin2_W):
    f32 = jnp.float32
    pad_idx = (jnp.arange(EPAD - E, dtype=jnp.int32) % (NPAD - N)) + N
    src2d = jnp.concatenate([edge_index[0], pad_idx]).reshape(EPAD // 128, 128)
    dst2d = jnp.concatenate([edge_index[1], pad_idx]).reshape(EPAD // 128, 128)
    zeros = jnp.zeros((NPAD, D), f32)
    A1 = jnp.stack([gat1_as, gat1_ad], axis=1)
    A2 = jnp.stack([gat2_as, gat2_ad], axis=1)
    bl = b_lin.reshape(1, HID)
    g1 = bn1_g.reshape(1, HID); b1 = bn1_b.reshape(1, HID)
    g2 = bn2_g.reshape(1, HID); b2 = bn2_b.reshape(1, HID)
    gb1 = gat1_b.reshape(1, HID); gb2 = gat2_b.reshape(1, HID)
    kf = k1.reshape(NHEAD * NCLU, HID)
    cw = conv1_w.reshape(1, NHEAD)
    batch2d = batch.reshape(N, 1)

    h0 = _tc_call(
        _linproj_body,
        jax.ShapeDtypeStruct((N, HID), f32),
        (x, W_lin, bl),
        grid=(NBLK,),
        in_specs=[pl.BlockSpec((BLK, 128), lambda i: (i, 0)),
                  _full((128, HID)), _full((1, HID))],
        out_specs=pl.BlockSpec((BLK, HID), lambda i: (i, 0)))

    hw48_1, svd1 = _tc_call(
        _prep_body,
        [jax.ShapeDtypeStruct((NPAD, D), f32),
         jax.ShapeDtypeStruct((NPAD, 2), f32)],
        (h0, g1, b1, gat1_W, A1))
    acc1 = _edge_call(hw48_1, svd1, src2d, dst2d, zeros)
    h1 = _tc_call(
        _combine_body,
        jax.ShapeDtypeStruct((N, HID), f32),
        (h0, hw48_1, svd1, acc1, gb1),
        grid=(NBLK,),
        in_specs=[pl.BlockSpec((BLK, HID), lambda i: (i, 0)),
                  pl.BlockSpec((BLK, D), lambda i: (i, 0)),
                  pl.BlockSpec((BLK, 2), lambda i: (i, 0)),
                  pl.BlockSpec((NC, BLK, D), lambda i: (0, i, 0)),
                  _full((1, HID))],
        out_specs=pl.BlockSpec((BLK, HID), lambda i: (i, 0)))

    hw48_2, svd2 = _tc_call(
        _prep_body,
        [jax.ShapeDtypeStruct((NPAD, D), f32),
         jax.ShapeDtypeStruct((NPAD, 2), f32)],
        (h1, g2, b2, gat2_W, A2))
    acc2 = _edge_call(hw48_2, svd2, src2d, dst2d, zeros)

    S, out_cs = _tc_call(
        _pool_body,
        [jax.ShapeDtypeStruct((N, NCLU), f32),
         jax.ShapeDtypeStruct((NG * NCLU, HID + 1), f32)],
        (h1, hw48_2, svd2, acc2, gb2, batch2d, kf, cw),
        grid=(NBLK,),
        in_specs=[pl.BlockSpec((BLK, HID), lambda i: (i, 0)),
                  pl.BlockSpec((BLK, D), lambda i: (i, 0)),
                  pl.BlockSpec((BLK, 2), lambda i: (i, 0)),
                  pl.BlockSpec((NC, BLK, D), lambda i: (0, i, 0)),
                  _full((1, HID)),
                  pl.BlockSpec((BLK, 1), lambda i: (i, 0)),
                  _full((NHEAD * NCLU, HID)),
                  _full((1, NHEAD))],
        out_specs=[pl.BlockSpec((BLK, NCLU), lambda i: (i, 0)),
                   _full((NG * NCLU, HID + 1))])

    logp, kl = _tc_call(
        _head_body,
        [jax.ShapeDtypeStruct((NG, NCLU), f32),
         jax.ShapeDtypeStruct((1, 1), f32)],
        (S, out_cs, batch2d, lin1_W, lin2_W))
    return logp, kl[0, 0]


# speedup vs baseline: 38.9092x; 1.1361x over previous
import dataclasses
import functools

import jax
import jax.numpy as jnp
from jax import lax
from jax.experimental import pallas as pl
from jax.experimental.pallas import tpu as pltpu
from jax.experimental.pallas import tpu_sc as plsc

N = 10000
E = 320000
NPAD = 10016
EPAD = 327680
D = 48
HID = 32
NG = 16
NCLU = 10
NHEAD = 5
NC, NS = 2, 16
NW = NC * NS
EPW = EPAD // NW
CH = 256
CROWS = CH // 128
NCHUNK = EPW // CH
BLK = 1000
NBLK = N // BLK
EPS = 1e-15
_PREC = lax.Precision.HIGHEST


def _leaky(x, s):
    return jnp.where(x > 0, x, s * x)


def _dotT(a, b):
    return lax.dot_general(a, b, (((0,), (0,)), ((), ())),
                           preferred_element_type=jnp.float32,
                           precision=_PREC)


def _dot(a, b):
    return jnp.dot(a, b, preferred_element_type=jnp.float32, precision=_PREC)


def _linproj_body(x_ref, wl_ref, bl_ref, h0_ref):
    h0_ref[...] = _dot(x_ref[...], wl_ref[...]) + bl_ref[...]


def _prep_body(h_ref, g_ref, b_ref, w_ref, a_ref, hw_ref, svd_ref):
    h = h_ref[...]
    m = jnp.mean(h, axis=0)
    v = jnp.mean((h - m) ** 2, axis=0)
    hb = _leaky((h - m) / jnp.sqrt(v + 1e-5) * g_ref[...] + b_ref[...], 0.01)
    hw = _dot(hb, w_ref[...])
    svd = _dot(hw, a_ref[...])
    hw48 = jnp.concatenate(
        [hw, jnp.ones((N, 1), jnp.float32),
         jnp.zeros((N, D - HID - 1), jnp.float32)], axis=1)
    hw_ref[...] = jnp.concatenate(
        [hw48, jnp.zeros((NPAD - N, D), jnp.float32)], axis=0)
    svd_ref[...] = jnp.concatenate(
        [svd, jnp.zeros((NPAD - N, 2), jnp.float32)], axis=0)


def _combine_vals(accv, hwv, svdv, biasv, hprev):
    accs = accv[0] + accv[1]
    z = svdv[:, 0:1] + svdv[:, 1:2]
    ws = jnp.exp(_leaky(z, 0.2))
    num = accs[:, :HID] + ws * hwv[:, :HID]
    den = accs[:, HID:HID + 1] + ws
    return hprev + num / den + biasv


def _combine_body(h_ref, hw_ref, svd_ref, acc_ref, bias_ref, out_ref):
    out_ref[...] = _combine_vals(acc_ref[...], hw_ref[...], svd_ref[...],
                                 bias_ref[...], h_ref[...])


def _pool_body(h_ref, hw_ref, svd_ref, acc_ref, bias_ref, batch_ref,
               kf_ref, cw_ref, s_ref, cs_ref):
    h2 = _combine_vals(acc_ref[...], hw_ref[...], svd_ref[...],
                       bias_ref[...], h_ref[...])
    kf = kf_ref[...]
    hn2 = jnp.sum(h2 * h2, axis=1, keepdims=True)
    kn2 = jnp.sum(kf * kf, axis=1)
    d2 = hn2 + kn2[None, :] - 2.0 * lax.dot_general(
        h2, kf, (((1,), (1,)), ((), ())),
        preferred_element_type=jnp.float32, precision=_PREC)
    d2 = jnp.maximum(d2, 0.0)
    dist = 1.0 / (1.0 + d2)
    i50h = lax.broadcasted_iota(jnp.int32, (NHEAD * NCLU, NHEAD), 0)
    i5h = lax.broadcasted_iota(jnp.int32, (NHEAD * NCLU, NHEAD), 1)
    M5 = (i50h // NCLU == i5h).astype(jnp.float32)
    dsum = _dot(dist, M5)
    dfull = lax.dot_general(dsum, M5, (((1,), (1,)), ((), ())),
                            preferred_element_type=jnp.float32,
                            precision=_PREC)
    distn = dist / dfull
    cwexp = lax.dot_general(cw_ref[...], M5, (((1,), (1,)), ((), ())),
                            preferred_element_type=jnp.float32,
                            precision=_PREC)
    i50k = lax.broadcasted_iota(jnp.int32, (NHEAD * NCLU, NCLU), 0)
    i10k = lax.broadcasted_iota(jnp.int32, (NHEAD * NCLU, NCLU), 1)
    M10 = (i50k % NCLU == i10k).astype(jnp.float32)
    S = _dot(distn * cwexp, M10)
    mx = jnp.max(S, axis=1, keepdims=True)
    e = jnp.exp(S - mx)
    S = e / jnp.sum(e, axis=1, keepdims=True)
    s_ref[...] = S
    bt = batch_ref[...]
    i16 = lax.broadcasted_iota(jnp.int32, (1, NG), 1)
    G = (bt == i16).astype(jnp.float32)
    iR0 = lax.broadcasted_iota(jnp.int32, (NG, NG * NCLU), 0)
    iR1 = lax.broadcasted_iota(jnp.int32, (NG, NG * NCLU), 1)
    R = (iR1 // NCLU == iR0).astype(jnp.float32)
    iT0 = lax.broadcasted_iota(jnp.int32, (NCLU, NG * NCLU), 0)
    iT1 = lax.broadcasted_iota(jnp.int32, (NCLU, NG * NCLU), 1)
    T = (iT1 % NCLU == iT0).astype(jnp.float32)
    P = _dot(G, R) * _dot(S, T)
    ext = jnp.concatenate([h2, jnp.ones((h2.shape[0], 1), jnp.float32)],
                          axis=1)

    @pl.when(pl.program_id(0) == 0)
    def _():
        cs_ref[...] = jnp.zeros_like(cs_ref)

    cs_ref[...] += _dotT(P, ext)


def _head_body(s_ref, cs_ref, batch_ref, lin1_ref, lin2_ref,
               logp_ref, kl_ref):
    out_cs = cs_ref[...]
    out1 = out_cs[:, :HID]
    colsum = out_cs[:, HID:HID + 1]
    iR0 = lax.broadcasted_iota(jnp.int32, (NG, NG * NCLU), 0)
    iR1 = lax.broadcasted_iota(jnp.int32, (NG, NG * NCLU), 1)
    R = (iR1 // NCLU == iR0).astype(jnp.float32)
    iT0 = lax.broadcasted_iota(jnp.int32, (NCLU, NG * NCLU), 0)
    iT1 = lax.broadcasted_iota(jnp.int32, (NCLU, NG * NCLU), 1)
    T = (iT1 % NCLU == iT0).astype(jnp.float32)
    x1 = _leaky(_dot(out1, lin1_ref[...]), 0.01)
    x2 = _dot(_dot(R, x1), lin2_ref[...])
    mx2 = jnp.max(x2, axis=1, keepdims=True)
    lse = jnp.log(jnp.sum(jnp.exp(x2 - mx2), axis=1, keepdims=True)) + mx2
    logp_ref[...] = x2 - lse
    cs16 = lax.dot_general(R * jnp.transpose(colsum), T,
                           (((1,), (1,)), ((), ())),
                           preferred_element_type=jnp.float32,
                           precision=_PREC)
    bt = batch_ref[...]
    i16 = lax.broadcasted_iota(jnp.int32, (1, NG), 1)
    G = (bt == i16).astype(jnp.float32)
    cs_pn = _dot(G, cs16)
    S = s_ref[...]
    Pn = (S * S) / jnp.maximum(cs_pn, EPS)
    Pn = Pn / jnp.sum(Pn, axis=1, keepdims=True)
    Pc = jnp.maximum(Pn, EPS)
    Sc = jnp.maximum(S, EPS)
    kl = jnp.sum(Pc * (jnp.log(Pc) - jnp.log(Sc))) / NG
    kl_ref[...] = jnp.reshape(kl, (1, 1))


def _edge_body(hw_hbm, svd_hbm, edge_hbm, zero_hbm, out_hbm,
               svd_v, idx0, idx1, rows, wbuf, acc, isem, gsem, ssem):
    cid = lax.axis_index("c")
    sid = lax.axis_index("s")
    wid = sid * NC + cid

    @pl.when(sid == 0)
    def _():
        pltpu.sync_copy(zero_hbm, acc)

    pltpu.sync_copy(svd_hbm, svd_v)
    plsc.subcore_barrier()

    col0 = lax.iota(jnp.int32, 16) * 0
    col1 = col0 + 1
    row0 = wid * (EPW // 128)
    idxb = (idx0, idx1)

    pltpu.async_copy(edge_hbm.at[pl.ds(row0, CROWS)], idx0, isem)

    def _half(k, par):
        c = 2 * k + par
        ib = idxb[par]
        nb = idxb[1 - par]
        r0 = row0 + c * CROWS
        pltpu.make_async_copy(edge_hbm.at[pl.ds(r0, CROWS)], ib, isem).wait()

        def _drain_prev():
            for j in range(CROWS):
                pltpu.make_async_copy(rows.at[pl.ds(j * 128, 128)],
                                      acc.at[nb.at[j, 1]], ssem).wait()
        if par == 1:
            _drain_prev()
        else:
            @pl.when(k >= 1)
            def _():
                _drain_prev()

        descs = [pltpu.async_copy(hw_hbm.at[ib.at[j, 0]],
                                  rows.at[pl.ds(j * 128, 128)], gsem)
                 for j in range(CROWS)]
        pltpu.async_copy(edge_hbm.at[pl.ds(r0 + CROWS, CROWS)], nb, isem)
        for j in range(CROWS):
            @pl.loop(0, 128 // 16)
            def _grp(g, j=j):
                s16 = ib[j, 0, pl.ds(g * 16, 16)]
                d16 = ib[j, 1, pl.ds(g * 16, 16)]
                z = (plsc.load_gather(svd_v, [s16, col0])
                     + plsc.load_gather(svd_v, [d16, col1]))
                z = jnp.where(z > 0, z, 0.2 * z)
                wbuf[pl.ds(j * 128 + g * 16, 16)] = jnp.exp(z)
        for dsc in descs:
            dsc.wait()

        @pl.loop(0, CH, unroll=4)
        def _mul(r):
            wv = plsc.load_gather(wbuf, [col0 + r])
            rows[r, pl.ds(0, 16)] = rows[r, pl.ds(0, 16)] * wv
            rows[r, pl.ds(16, 16)] = rows[r, pl.ds(16, 16)] * wv
            rows[r, pl.ds(32, 16)] = rows[r, pl.ds(32, 16)] * wv

        for j in range(CROWS):
            pltpu.async_copy(rows.at[pl.ds(j * 128, 128)],
                             acc.at[ib.at[j, 1]], ssem, add=True)

    @pl.loop(0, NCHUNK // 2)
    def _k(k):
        _half(k, 0)
        _half(k, 1)

    for j in range(CROWS):
        pltpu.make_async_copy(rows.at[pl.ds(j * 128, 128)],
                              acc.at[idx1.at[j, 1]], ssem).wait()
    pltpu.make_async_copy(edge_hbm.at[pl.ds(row0, CROWS)], idx0, isem).wait()

    plsc.subcore_barrier()

    @pl.when(sid == 0)
    def _():
        pltpu.sync_copy(acc, out_hbm.at[cid])


def _edge_call(hw48, svd, edges, zeros, interpret=False):
    mesh = plsc.VectorSubcoreMesh(core_axis_name="c", subcore_axis_name="s",
                                  num_cores=NC, num_subcores=NS)
    cp = pltpu.CompilerParams(use_tc_tiling_on_sc=False)
    if "needs_layout_passes" in pltpu.CompilerParams.__dataclass_fields__:
        cp = dataclasses.replace(cp, needs_layout_passes=False)
    kern = pl.kernel(
        _edge_body,
        out_type=jax.ShapeDtypeStruct((NC, NPAD, D), jnp.float32),
        mesh=mesh,
        scratch_types=[
            pltpu.VMEM((NPAD, 2), jnp.float32),
            pltpu.VMEM((CROWS, 2, 128), jnp.int32),
            pltpu.VMEM((CROWS, 2, 128), jnp.int32),
            pltpu.VMEM((CH, D), jnp.float32),
            pltpu.VMEM((CH,), jnp.float32),
            pltpu.VMEM_SHARED((NPAD, D), jnp.float32),
            pltpu.SemaphoreType.DMA,
            pltpu.SemaphoreType.DMA,
            pltpu.SemaphoreType.DMA,
        ],
        interpret=interpret,
        compiler_params=cp,
    )
    return kern(hw48, svd, edges, zeros)


def _tc_call(body, out_shapes, args, grid=None, in_specs=None, out_specs=None,
             interpret=False):
    kwargs = {}
    if grid is not None:
        kwargs = dict(grid=grid, in_specs=in_specs, out_specs=out_specs)
    return pl.pallas_call(
        body,
        out_shape=out_shapes,
        interpret=interpret,
        **kwargs,
    )(*args)


def _full(shape):
    return pl.BlockSpec(shape, lambda i: tuple(0 for _ in shape))


def kernel(x, edge_index, batch, W_lin, b_lin, bn1_g, bn1_b, gat1_W, gat1_as,
           gat1_ad, gat1_b, bn2_g, bn2_b, gat2_W, gat2_as, gat2_ad, gat2_b,
           k1, conv1_w, lin1_W, k2, conv2_w, lin2_W):
    f32 = jnp.float32
    pad_idx = (jnp.arange(EPAD - E, dtype=jnp.int32) % (NPAD - N)) + N
    src2d = jnp.concatenate([edge_index[0], pad_idx]).reshape(EPAD // 128, 128)
    dst2d = jnp.concatenate([edge_index[1], pad_idx]).reshape(EPAD // 128, 128)
    edges = jnp.stack([src2d, dst2d], axis=1)
    edges = jnp.concatenate(
        [edges, jnp.full((CROWS, 2, 128), N, jnp.int32)], axis=0)
    zeros = jnp.zeros((NPAD, D), f32)
    A1 = jnp.stack([gat1_as, gat1_ad], axis=1)
    A2 = jnp.stack([gat2_as, gat2_ad], axis=1)
    bl = b_lin.reshape(1, HID)
    g1 = bn1_g.reshape(1, HID); b1 = bn1_b.reshape(1, HID)
    g2 = bn2_g.reshape(1, HID); b2 = bn2_b.reshape(1, HID)
    gb1 = gat1_b.reshape(1, HID); gb2 = gat2_b.reshape(1, HID)
    kf = k1.reshape(NHEAD * NCLU, HID)
    cw = conv1_w.reshape(1, NHEAD)
    batch2d = batch.reshape(N, 1)

    h0 = _tc_call(
        _linproj_body,
        jax.ShapeDtypeStruct((N, HID), f32),
        (x, W_lin, bl),
        grid=(NBLK,),
        in_specs=[pl.BlockSpec((BLK, 128), lambda i: (i, 0)),
                  _full((128, HID)), _full((1, HID))],
        out_specs=pl.BlockSpec((BLK, HID), lambda i: (i, 0)))

    hw48_1, svd1 = _tc_call(
        _prep_body,
        [jax.ShapeDtypeStruct((NPAD, D), f32),
         jax.ShapeDtypeStruct((NPAD, 2), f32)],
        (h0, g1, b1, gat1_W, A1))
    acc1 = _edge_call(hw48_1, svd1, edges, zeros)
    h1 = _tc_call(
        _combine_body,
        jax.ShapeDtypeStruct((N, HID), f32),
        (h0, hw48_1, svd1, acc1, gb1),
        grid=(NBLK,),
        in_specs=[pl.BlockSpec((BLK, HID), lambda i: (i, 0)),
                  pl.BlockSpec((BLK, D), lambda i: (i, 0)),
                  pl.BlockSpec((BLK, 2), lambda i: (i, 0)),
                  pl.BlockSpec((NC, BLK, D), lambda i: (0, i, 0)),
                  _full((1, HID))],
        out_specs=pl.BlockSpec((BLK, HID), lambda i: (i, 0)))

    hw48_2, svd2 = _tc_call(
        _prep_body,
        [jax.ShapeDtypeStruct((NPAD, D), f32),
         jax.ShapeDtypeStruct((NPAD, 2), f32)],
        (h1, g2, b2, gat2_W, A2))
    acc2 = _edge_call(hw48_2, svd2, edges, zeros)

    S, out_cs = _tc_call(
        _pool_body,
        [jax.ShapeDtypeStruct((N, NCLU), f32),
         jax.ShapeDtypeStruct((NG * NCLU, HID + 1), f32)],
        (h1, hw48_2, svd2, acc2, gb2, batch2d, kf, cw),
        grid=(NBLK,),
        in_specs=[pl.BlockSpec((BLK, HID), lambda i: (i, 0)),
                  pl.BlockSpec((BLK, D), lambda i: (i, 0)),
                  pl.BlockSpec((BLK, 2), lambda i: (i, 0)),
                  pl.BlockSpec((NC, BLK, D), lambda i: (0, i, 0)),
                  _full((1, HID)),
                  pl.BlockSpec((BLK, 1), lambda i: (i, 0)),
                  _full((NHEAD * NCLU, HID)),
                  _full((1, NHEAD))],
        out_specs=[pl.BlockSpec((BLK, NCLU), lambda i: (i, 0)),
                   _full((NG * NCLU, HID + 1))])

    logp, kl = _tc_call(
        _head_body,
        [jax.ShapeDtypeStruct((NG, NCLU), f32),
         jax.ShapeDtypeStruct((1, 1), f32)],
        (S, out_cs, batch2d, lin1_W, lin2_W))
    return logp, kl[0, 0]


# speedup vs baseline: 40.7031x; 1.0461x over previous
import dataclasses
import functools

import jax
import jax.numpy as jnp
from jax import lax
from jax.experimental import pallas as pl
from jax.experimental.pallas import tpu as pltpu
from jax.experimental.pallas import tpu_sc as plsc

N = 10000
E = 320000
NPAD = 10016
EPAD = 327680
D = 32
HID = 32
NG = 16
NCLU = 10
NHEAD = 5
NC, NS = 2, 16
NW = NC * NS
EPW = EPAD // NW
CH = 256
CROWS = CH // 128
NCHUNK = EPW // CH
BLK = 1000
NBLK = N // BLK
EPS = 1e-15
_PREC = lax.Precision.HIGHEST


def _leaky(x, s):
    return jnp.where(x > 0, x, s * x)


def _dotT(a, b):
    return lax.dot_general(a, b, (((0,), (0,)), ((), ())),
                           preferred_element_type=jnp.float32,
                           precision=_PREC)


def _dot(a, b):
    return jnp.dot(a, b, preferred_element_type=jnp.float32, precision=_PREC)


def _linproj_body(x_ref, wl_ref, bl_ref, h0_ref):
    h0_ref[...] = _dot(x_ref[...], wl_ref[...]) + bl_ref[...]


def _prep_body(h_ref, g_ref, b_ref, w_ref, a_ref, hw_ref, svd_ref):
    h = h_ref[...]
    m = jnp.mean(h, axis=0)
    v = jnp.mean((h - m) ** 2, axis=0)
    hb = _leaky((h - m) / jnp.sqrt(v + 1e-5) * g_ref[...] + b_ref[...], 0.01)
    hw = _dot(hb, w_ref[...])
    svd = _dot(hw, a_ref[...])
    hw_ref[...] = jnp.concatenate(
        [hw, jnp.zeros((NPAD - N, D), jnp.float32)], axis=0)
    svd_ref[...] = jnp.concatenate(
        [svd, jnp.zeros((NPAD - N, 2), jnp.float32)], axis=0)


def _denred_body(den_ref, out_ref):
    out_ref[...] = _dotT(den_ref[...], jnp.ones((NW, 1), jnp.float32))


def _combine_vals(accv, dtot, hwv, svdv, biasv, hprev):
    accs = accv[0] + accv[1]
    z = svdv[:, 0:1] + svdv[:, 1:2]
    ws = jnp.exp(_leaky(z, 0.2))
    num = accs + ws * hwv
    den = dtot + ws
    return hprev + num / den + biasv


def _combine_body(h_ref, hw_ref, svd_ref, acc_ref, den_ref, bias_ref,
                  out_ref):
    out_ref[...] = _combine_vals(acc_ref[...], den_ref[...], hw_ref[...],
                                 svd_ref[...], bias_ref[...], h_ref[...])


def _pool_body(h_ref, hw_ref, svd_ref, acc_ref, den_ref, bias_ref,
               batch_ref, kf_ref, cw_ref, s_ref, cs_ref):
    h2 = _combine_vals(acc_ref[...], den_ref[...], hw_ref[...], svd_ref[...],
                       bias_ref[...], h_ref[...])
    kf = kf_ref[...]
    hn2 = jnp.sum(h2 * h2, axis=1, keepdims=True)
    kn2 = jnp.sum(kf * kf, axis=1)
    d2 = hn2 + kn2[None, :] - 2.0 * lax.dot_general(
        h2, kf, (((1,), (1,)), ((), ())),
        preferred_element_type=jnp.float32, precision=_PREC)
    d2 = jnp.maximum(d2, 0.0)
    dist = 1.0 / (1.0 + d2)
    i50h = lax.broadcasted_iota(jnp.int32, (NHEAD * NCLU, NHEAD), 0)
    i5h = lax.broadcasted_iota(jnp.int32, (NHEAD * NCLU, NHEAD), 1)
    M5 = (i50h // NCLU == i5h).astype(jnp.float32)
    dsum = _dot(dist, M5)
    dfull = lax.dot_general(dsum, M5, (((1,), (1,)), ((), ())),
                            preferred_element_type=jnp.float32,
                            precision=_PREC)
    distn = dist / dfull
    cwexp = lax.dot_general(cw_ref[...], M5, (((1,), (1,)), ((), ())),
                            preferred_element_type=jnp.float32,
                            precision=_PREC)
    i50k = lax.broadcasted_iota(jnp.int32, (NHEAD * NCLU, NCLU), 0)
    i10k = lax.broadcasted_iota(jnp.int32, (NHEAD * NCLU, NCLU), 1)
    M10 = (i50k % NCLU == i10k).astype(jnp.float32)
    S = _dot(distn * cwexp, M10)
    mx = jnp.max(S, axis=1, keepdims=True)
    e = jnp.exp(S - mx)
    S = e / jnp.sum(e, axis=1, keepdims=True)
    s_ref[...] = S
    bt = batch_ref[...]
    i16 = lax.broadcasted_iota(jnp.int32, (1, NG), 1)
    G = (bt == i16).astype(jnp.float32)
    iR0 = lax.broadcasted_iota(jnp.int32, (NG, NG * NCLU), 0)
    iR1 = lax.broadcasted_iota(jnp.int32, (NG, NG * NCLU), 1)
    R = (iR1 // NCLU == iR0).astype(jnp.float32)
    iT0 = lax.broadcasted_iota(jnp.int32, (NCLU, NG * NCLU), 0)
    iT1 = lax.broadcasted_iota(jnp.int32, (NCLU, NG * NCLU), 1)
    T = (iT1 % NCLU == iT0).astype(jnp.float32)
    P = _dot(G, R) * _dot(S, T)
    ext = jnp.concatenate([h2, jnp.ones((h2.shape[0], 1), jnp.float32)],
                          axis=1)

    @pl.when(pl.program_id(0) == 0)
    def _():
        cs_ref[...] = jnp.zeros_like(cs_ref)

    cs_ref[...] += _dotT(P, ext)


def _head_body(s_ref, cs_ref, batch_ref, lin1_ref, lin2_ref,
               logp_ref, kl_ref):
    out_cs = cs_ref[...]
    out1 = out_cs[:, :HID]
    colsum = out_cs[:, HID:HID + 1]
    iR0 = lax.broadcasted_iota(jnp.int32, (NG, NG * NCLU), 0)
    iR1 = lax.broadcasted_iota(jnp.int32, (NG, NG * NCLU), 1)
    R = (iR1 // NCLU == iR0).astype(jnp.float32)
    iT0 = lax.broadcasted_iota(jnp.int32, (NCLU, NG * NCLU), 0)
    iT1 = lax.broadcasted_iota(jnp.int32, (NCLU, NG * NCLU), 1)
    T = (iT1 % NCLU == iT0).astype(jnp.float32)
    x1 = _leaky(_dot(out1, lin1_ref[...]), 0.01)
    x2 = _dot(_dot(R, x1), lin2_ref[...])
    mx2 = jnp.max(x2, axis=1, keepdims=True)
    lse = jnp.log(jnp.sum(jnp.exp(x2 - mx2), axis=1, keepdims=True)) + mx2
    logp_ref[...] = x2 - lse
    cs16 = lax.dot_general(R * jnp.transpose(colsum), T,
                           (((1,), (1,)), ((), ())),
                           preferred_element_type=jnp.float32,
                           precision=_PREC)
    bt = batch_ref[...]
    i16 = lax.broadcasted_iota(jnp.int32, (1, NG), 1)
    G = (bt == i16).astype(jnp.float32)
    cs_pn = _dot(G, cs16)
    S = s_ref[...]
    Pn = (S * S) / jnp.maximum(cs_pn, EPS)
    Pn = Pn / jnp.sum(Pn, axis=1, keepdims=True)
    Pc = jnp.maximum(Pn, EPS)
    Sc = jnp.maximum(S, EPS)
    kl = jnp.sum(Pc * (jnp.log(Pc) - jnp.log(Sc))) / NG
    kl_ref[...] = jnp.reshape(kl, (1, 1))


def _edge_body(hw_hbm, svd_hbm, edge_hbm, zero_hbm, out_hbm, den_hbm,
               svd_v, den_v, idx0, idx1, rows0, rows1, wbuf, acc,
               isem, gsem, ssem):
    cid = lax.axis_index("c")
    sid = lax.axis_index("s")
    wid = sid * NC + cid

    @pl.when(sid == 0)
    def _():
        pltpu.sync_copy(zero_hbm, acc)

    pltpu.sync_copy(svd_hbm, svd_v)
    z16 = jnp.zeros((16,), jnp.float32)

    @pl.loop(0, NPAD // 16)
    def _zero_den(i):
        den_v[pl.ds(i * 16, 16)] = z16

    plsc.subcore_barrier()

    col0 = lax.iota(jnp.int32, 16) * 0
    col1 = col0 + 1
    row0 = wid * (EPW // 128)
    idxb = (idx0, idx1)
    rowsb = (rows0, rows1)

    pltpu.async_copy(edge_hbm.at[pl.ds(row0, CROWS)], idx0, isem)

    def _half(k, par):
        c = 2 * k + par
        ib, rb = idxb[par], rowsb[par]
        nb, prb = idxb[1 - par], rowsb[1 - par]
        r0 = row0 + c * CROWS
        pltpu.make_async_copy(edge_hbm.at[pl.ds(r0, CROWS)], ib, isem).wait()
        descs = [pltpu.async_copy(hw_hbm.at[ib.at[j, 0]],
                                  rb.at[pl.ds(j * 128, 128)], gsem)
                 for j in range(CROWS)]

        def _drain_prev():
            for j in range(CROWS):
                pltpu.make_async_copy(prb.at[pl.ds(j * 128, 128)],
                                      acc.at[nb.at[j, 1]], ssem).wait()
        if par == 1:
            _drain_prev()
        else:
            @pl.when(k >= 1)
            def _():
                _drain_prev()

        pltpu.async_copy(edge_hbm.at[pl.ds(r0 + CROWS, CROWS)], nb, isem)
        for j in range(CROWS):
            @pl.loop(0, 128 // 16)
            def _grp(g, j=j):
                s16 = ib[j, 0, pl.ds(g * 16, 16)]
                d16 = ib[j, 1, pl.ds(g * 16, 16)]
                z = (plsc.load_gather(svd_v, [s16, col0])
                     + plsc.load_gather(svd_v, [d16, col1]))
                z = jnp.where(z > 0, z, 0.2 * z)
                w = jnp.exp(z)
                wbuf[pl.ds(j * 128 + g * 16, 16)] = w
                plsc.addupdate_scatter(den_v, [d16], w)
        for dsc in descs:
            dsc.wait()

        @pl.loop(0, CH, unroll=4)
        def _mul(r):
            wv = plsc.load_gather(wbuf, [col0 + r])
            rb[r, pl.ds(0, 16)] = rb[r, pl.ds(0, 16)] * wv
            rb[r, pl.ds(16, 16)] = rb[r, pl.ds(16, 16)] * wv

        for j in range(CROWS):
            pltpu.async_copy(rb.at[pl.ds(j * 128, 128)],
                             acc.at[ib.at[j, 1]], ssem, add=True)

    @pl.loop(0, NCHUNK // 2)
    def _k(k):
        _half(k, 0)
        _half(k, 1)

    for j in range(CROWS):
        pltpu.make_async_copy(rows1.at[pl.ds(j * 128, 128)],
                              acc.at[idx1.at[j, 1]], ssem).wait()
    pltpu.make_async_copy(edge_hbm.at[pl.ds(row0, CROWS)], idx0, isem).wait()

    pltpu.sync_copy(den_v, den_hbm.at[wid])
    plsc.subcore_barrier()

    @pl.when(sid == 0)
    def _():
        pltpu.sync_copy(acc, out_hbm.at[cid])


def _edge_call(hw, svd, edges, zeros, interpret=False):
    mesh = plsc.VectorSubcoreMesh(core_axis_name="c", subcore_axis_name="s",
                                  num_cores=NC, num_subcores=NS)
    cp = pltpu.CompilerParams(use_tc_tiling_on_sc=False)
    if "needs_layout_passes" in pltpu.CompilerParams.__dataclass_fields__:
        cp = dataclasses.replace(cp, needs_layout_passes=False)
    kern = pl.kernel(
        _edge_body,
        out_type=[jax.ShapeDtypeStruct((NC, NPAD, D), jnp.float32),
                  jax.ShapeDtypeStruct((NW, NPAD), jnp.float32)],
        mesh=mesh,
        scratch_types=[
            pltpu.VMEM((NPAD, 2), jnp.float32),
            pltpu.VMEM((NPAD,), jnp.float32),
            pltpu.VMEM((CROWS, 2, 128), jnp.int32),
            pltpu.VMEM((CROWS, 2, 128), jnp.int32),
            pltpu.VMEM((CH, D), jnp.float32),
            pltpu.VMEM((CH, D), jnp.float32),
            pltpu.VMEM((CH,), jnp.float32),
            pltpu.VMEM_SHARED((NPAD, D), jnp.float32),
            pltpu.SemaphoreType.DMA,
            pltpu.SemaphoreType.DMA,
            pltpu.SemaphoreType.DMA,
        ],
        interpret=interpret,
        compiler_params=cp,
    )
    return kern(hw, svd, edges, zeros)


def _tc_call(body, out_shapes, args, grid=None, in_specs=None, out_specs=None,
             interpret=False):
    kwargs = {}
    if grid is not None:
        kwargs = dict(grid=grid, in_specs=in_specs, out_specs=out_specs)
    return pl.pallas_call(
        body,
        out_shape=out_shapes,
        interpret=interpret,
        **kwargs,
    )(*args)


def _full(shape):
    return pl.BlockSpec(shape, lambda i: tuple(0 for _ in shape))


def kernel(x, edge_index, batch, W_lin, b_lin, bn1_g, bn1_b, gat1_W, gat1_as,
           gat1_ad, gat1_b, bn2_g, bn2_b, gat2_W, gat2_as, gat2_ad, gat2_b,
           k1, conv1_w, lin1_W, k2, conv2_w, lin2_W):
    f32 = jnp.float32
    pad_idx = (jnp.arange(EPAD - E, dtype=jnp.int32) % (NPAD - N)) + N
    src2d = jnp.concatenate([edge_index[0], pad_idx]).reshape(EPAD // 128, 128)
    dst2d = jnp.concatenate([edge_index[1], pad_idx]).reshape(EPAD // 128, 128)
    edges = jnp.stack([src2d, dst2d], axis=1)
    edges = jnp.concatenate(
        [edges, jnp.full((CROWS, 2, 128), N, jnp.int32)], axis=0)
    zeros = jnp.zeros((NPAD, D), f32)
    A1 = jnp.stack([gat1_as, gat1_ad], axis=1)
    A2 = jnp.stack([gat2_as, gat2_ad], axis=1)
    bl = b_lin.reshape(1, HID)
    g1 = bn1_g.reshape(1, HID); b1 = bn1_b.reshape(1, HID)
    g2 = bn2_g.reshape(1, HID); b2 = bn2_b.reshape(1, HID)
    gb1 = gat1_b.reshape(1, HID); gb2 = gat2_b.reshape(1, HID)
    kf = k1.reshape(NHEAD * NCLU, HID)
    cw = conv1_w.reshape(1, NHEAD)
    batch2d = batch.reshape(N, 1)

    h0 = _tc_call(
        _linproj_body,
        jax.ShapeDtypeStruct((N, HID), f32),
        (x, W_lin, bl),
        grid=(NBLK,),
        in_specs=[pl.BlockSpec((BLK, 128), lambda i: (i, 0)),
                  _full((128, HID)), _full((1, HID))],
        out_specs=pl.BlockSpec((BLK, HID), lambda i: (i, 0)))

    hw48_1, svd1 = _tc_call(
        _prep_body,
        [jax.ShapeDtypeStruct((NPAD, D), f32),
         jax.ShapeDtypeStruct((NPAD, 2), f32)],
        (h0, g1, b1, gat1_W, A1))
    acc1, den1 = _edge_call(hw48_1, svd1, edges, zeros)
    dtot1 = _tc_call(_denred_body, jax.ShapeDtypeStruct((NPAD, 1), f32),
                     (den1,))
    h1 = _tc_call(
        _combine_body,
        jax.ShapeDtypeStruct((N, HID), f32),
        (h0, hw48_1, svd1, acc1, dtot1, gb1),
        grid=(NBLK,),
        in_specs=[pl.BlockSpec((BLK, HID), lambda i: (i, 0)),
                  pl.BlockSpec((BLK, D), lambda i: (i, 0)),
                  pl.BlockSpec((BLK, 2), lambda i: (i, 0)),
                  pl.BlockSpec((NC, BLK, D), lambda i: (0, i, 0)),
                  pl.BlockSpec((BLK, 1), lambda i: (i, 0)),
                  _full((1, HID))],
        out_specs=pl.BlockSpec((BLK, HID), lambda i: (i, 0)))

    hw48_2, svd2 = _tc_call(
        _prep_body,
        [jax.ShapeDtypeStruct((NPAD, D), f32),
         jax.ShapeDtypeStruct((NPAD, 2), f32)],
        (h1, g2, b2, gat2_W, A2))
    acc2, den2 = _edge_call(hw48_2, svd2, edges, zeros)
    dtot2 = _tc_call(_denred_body, jax.ShapeDtypeStruct((NPAD, 1), f32),
                     (den2,))

    S, out_cs = _tc_call(
        _pool_body,
        [jax.ShapeDtypeStruct((N, NCLU), f32),
         jax.ShapeDtypeStruct((NG * NCLU, HID + 1), f32)],
        (h1, hw48_2, svd2, acc2, dtot2, gb2, batch2d, kf, cw),
        grid=(NBLK,),
        in_specs=[pl.BlockSpec((BLK, HID), lambda i: (i, 0)),
                  pl.BlockSpec((BLK, D), lambda i: (i, 0)),
                  pl.BlockSpec((BLK, 2), lambda i: (i, 0)),
                  pl.BlockSpec((NC, BLK, D), lambda i: (0, i, 0)),
                  pl.BlockSpec((BLK, 1), lambda i: (i, 0)),
                  _full((1, HID)),
                  pl.BlockSpec((BLK, 1), lambda i: (i, 0)),
                  _full((NHEAD * NCLU, HID)),
                  _full((1, NHEAD))],
        out_specs=[pl.BlockSpec((BLK, NCLU), lambda i: (i, 0)),
                   _full((NG * NCLU, HID + 1))])

    logp, kl = _tc_call(
        _head_body,
        [jax.ShapeDtypeStruct((NG, NCLU), f32),
         jax.ShapeDtypeStruct((1, 1), f32)],
        (S, out_cs, batch2d, lin1_W, lin2_W))
    return logp, kl[0, 0]


# speedup vs baseline: 47.0377x; 1.1556x over previous
import dataclasses
import functools

import jax
import jax.numpy as jnp
from jax import lax
from jax.experimental import pallas as pl
from jax.experimental.pallas import tpu as pltpu
from jax.experimental.pallas import tpu_sc as plsc

N = 10000
E = 320000
NPAD = 10016
EPAD = 327680
D = 32
HID = 32
NG = 16
NCLU = 10
NHEAD = 5
NC, NS = 2, 16
NW = NC * NS
EPW = EPAD // NW
CH = 128
CROWS = CH // 128
NCHUNK = EPW // CH
BLK = 1000
NBLK = N // BLK
EPS = 1e-15
_PREC = lax.Precision.HIGHEST


def _leaky(x, s):
    return jnp.where(x > 0, x, s * x)


def _dotT(a, b):
    return lax.dot_general(a, b, (((0,), (0,)), ((), ())),
                           preferred_element_type=jnp.float32,
                           precision=_PREC)


def _dot(a, b):
    return jnp.dot(a, b, preferred_element_type=jnp.float32, precision=_PREC)


def _linproj_body(x_ref, wl_ref, bl_ref, h0_ref):
    h0_ref[...] = _dot(x_ref[...], wl_ref[...]) + bl_ref[...]


def _prep_body(h_ref, g_ref, b_ref, w_ref, a_ref, hw_ref, svd_ref):
    h = h_ref[...]
    m = jnp.mean(h, axis=0)
    v = jnp.mean((h - m) ** 2, axis=0)
    hb = _leaky((h - m) / jnp.sqrt(v + 1e-5) * g_ref[...] + b_ref[...], 0.01)
    hw = _dot(hb, w_ref[...])
    svd = _dot(hw, a_ref[...])
    hw_ref[...] = jnp.concatenate(
        [hw, jnp.zeros((NPAD - N, D), jnp.float32)], axis=0)
    svd_ref[...] = jnp.concatenate(
        [svd, jnp.zeros((NPAD - N, 2), jnp.float32)], axis=0)


def _denred_body(den_ref, out_ref):
    out_ref[...] = _dotT(den_ref[...], jnp.ones((NW, 1), jnp.float32))


def _combine_vals(accv, dtot, hwv, svdv, biasv, hprev):
    accs = accv[0] + accv[1]
    z = svdv[:, 0:1] + svdv[:, 1:2]
    ws = jnp.exp(_leaky(z, 0.2))
    num = accs + ws * hwv
    den = dtot + ws
    return hprev + num / den + biasv


def _combine_body(h_ref, hw_ref, svd_ref, acc_ref, den_ref, bias_ref,
                  out_ref):
    out_ref[...] = _combine_vals(acc_ref[...], den_ref[...], hw_ref[...],
                                 svd_ref[...], bias_ref[...], h_ref[...])


def _pool_body(h_ref, hw_ref, svd_ref, acc_ref, den_ref, bias_ref,
               batch_ref, kf_ref, cw_ref, s_ref, cs_ref):
    h2 = _combine_vals(acc_ref[...], den_ref[...], hw_ref[...], svd_ref[...],
                       bias_ref[...], h_ref[...])
    kf = kf_ref[...]
    hn2 = jnp.sum(h2 * h2, axis=1, keepdims=True)
    kn2 = jnp.sum(kf * kf, axis=1)
    d2 = hn2 + kn2[None, :] - 2.0 * lax.dot_general(
        h2, kf, (((1,), (1,)), ((), ())),
        preferred_element_type=jnp.float32, precision=_PREC)
    d2 = jnp.maximum(d2, 0.0)
    dist = 1.0 / (1.0 + d2)
    i50h = lax.broadcasted_iota(jnp.int32, (NHEAD * NCLU, NHEAD), 0)
    i5h = lax.broadcasted_iota(jnp.int32, (NHEAD * NCLU, NHEAD), 1)
    M5 = (i50h // NCLU == i5h).astype(jnp.float32)
    dsum = _dot(dist, M5)
    dfull = lax.dot_general(dsum, M5, (((1,), (1,)), ((), ())),
                            preferred_element_type=jnp.float32,
                            precision=_PREC)
    distn = dist / dfull
    cwexp = lax.dot_general(cw_ref[...], M5, (((1,), (1,)), ((), ())),
                            preferred_element_type=jnp.float32,
                            precision=_PREC)
    i50k = lax.broadcasted_iota(jnp.int32, (NHEAD * NCLU, NCLU), 0)
    i10k = lax.broadcasted_iota(jnp.int32, (NHEAD * NCLU, NCLU), 1)
    M10 = (i50k % NCLU == i10k).astype(jnp.float32)
    S = _dot(distn * cwexp, M10)
    mx = jnp.max(S, axis=1, keepdims=True)
    e = jnp.exp(S - mx)
    S = e / jnp.sum(e, axis=1, keepdims=True)
    s_ref[...] = S
    bt = batch_ref[...]
    i16 = lax.broadcasted_iota(jnp.int32, (1, NG), 1)
    G = (bt == i16).astype(jnp.float32)
    iR0 = lax.broadcasted_iota(jnp.int32, (NG, NG * NCLU), 0)
    iR1 = lax.broadcasted_iota(jnp.int32, (NG, NG * NCLU), 1)
    R = (iR1 // NCLU == iR0).astype(jnp.float32)
    iT0 = lax.broadcasted_iota(jnp.int32, (NCLU, NG * NCLU), 0)
    iT1 = lax.broadcasted_iota(jnp.int32, (NCLU, NG * NCLU), 1)
    T = (iT1 % NCLU == iT0).astype(jnp.float32)
    P = _dot(G, R) * _dot(S, T)
    ext = jnp.concatenate([h2, jnp.ones((h2.shape[0], 1), jnp.float32)],
                          axis=1)

    @pl.when(pl.program_id(0) == 0)
    def _():
        cs_ref[...] = jnp.zeros_like(cs_ref)

    cs_ref[...] += _dotT(P, ext)


def _head_body(s_ref, cs_ref, batch_ref, lin1_ref, lin2_ref,
               logp_ref, kl_ref):
    out_cs = cs_ref[...]
    out1 = out_cs[:, :HID]
    colsum = out_cs[:, HID:HID + 1]
    iR0 = lax.broadcasted_iota(jnp.int32, (NG, NG * NCLU), 0)
    iR1 = lax.broadcasted_iota(jnp.int32, (NG, NG * NCLU), 1)
    R = (iR1 // NCLU == iR0).astype(jnp.float32)
    iT0 = lax.broadcasted_iota(jnp.int32, (NCLU, NG * NCLU), 0)
    iT1 = lax.broadcasted_iota(jnp.int32, (NCLU, NG * NCLU), 1)
    T = (iT1 % NCLU == iT0).astype(jnp.float32)
    x1 = _leaky(_dot(out1, lin1_ref[...]), 0.01)
    x2 = _dot(_dot(R, x1), lin2_ref[...])
    mx2 = jnp.max(x2, axis=1, keepdims=True)
    lse = jnp.log(jnp.sum(jnp.exp(x2 - mx2), axis=1, keepdims=True)) + mx2
    logp_ref[...] = x2 - lse
    cs16 = lax.dot_general(R * jnp.transpose(colsum), T,
                           (((1,), (1,)), ((), ())),
                           preferred_element_type=jnp.float32,
                           precision=_PREC)
    bt = batch_ref[...]
    i16 = lax.broadcasted_iota(jnp.int32, (1, NG), 1)
    G = (bt == i16).astype(jnp.float32)
    cs_pn = _dot(G, cs16)
    S = s_ref[...]
    Pn = (S * S) / jnp.maximum(cs_pn, EPS)
    Pn = Pn / jnp.sum(Pn, axis=1, keepdims=True)
    Pc = jnp.maximum(Pn, EPS)
    Sc = jnp.maximum(S, EPS)
    kl = jnp.sum(Pc * (jnp.log(Pc) - jnp.log(Sc))) / NG
    kl_ref[...] = jnp.reshape(kl, (1, 1))


def _edge_body(hw_hbm, svd_hbm, edge_hbm, zero_hbm, out_hbm, den_hbm,
               svd_v, den_v, i0, i1, i2, i3, r0, r1, r2, r3, w0, w1, acc,
               isem, gsem, ssem):
    cid = lax.axis_index("c")
    sid = lax.axis_index("s")
    wid = sid * NC + cid

    @pl.when(sid == 0)
    def _():
        pltpu.sync_copy(zero_hbm, acc)

    pltpu.sync_copy(svd_hbm, svd_v)
    z16 = jnp.zeros((16,), jnp.float32)

    @pl.loop(0, NPAD // 16)
    def _zero_den(i):
        den_v[pl.ds(i * 16, 16)] = z16

    plsc.subcore_barrier()

    col0 = lax.iota(jnp.int32, 16) * 0
    col1 = col0 + 1
    row0 = wid * (EPW // 128)
    idxq = (i0, i1, i2, i3)
    rowsq = (r0, r1, r2, r3)
    wq = (w0, w1)

    def _idx_fire(x, slot):
        pltpu.async_copy(edge_hbm.at[pl.ds(row0 + x * CROWS, CROWS)],
                         idxq[slot], isem.at[slot])

    def _idx_wait(x, slot):
        pltpu.make_async_copy(edge_hbm.at[pl.ds(row0 + x * CROWS, CROWS)],
                              idxq[slot], isem.at[slot]).wait()

    def _gat_fire(slot):
        ib, rb = idxq[slot], rowsq[slot]
        for j in range(CROWS):
            pltpu.async_copy(hw_hbm.at[ib.at[j, 0]],
                             rb.at[pl.ds(j * 128, 128)], gsem.at[slot])

    def _gat_wait(slot):
        ib, rb = idxq[slot], rowsq[slot]
        for j in range(CROWS):
            pltpu.make_async_copy(hw_hbm.at[ib.at[j, 0]],
                                  rb.at[pl.ds(j * 128, 128)],
                                  gsem.at[slot]).wait()

    def _sc_fire(slot):
        ib, rb = idxq[slot], rowsq[slot]
        for j in range(CROWS):
            pltpu.async_copy(rb.at[pl.ds(j * 128, 128)],
                             acc.at[ib.at[j, 1]], ssem.at[slot], add=True)

    def _sc_drain(slot):
        ib, rb = idxq[slot], rowsq[slot]
        for j in range(CROWS):
            pltpu.make_async_copy(rb.at[pl.ds(j * 128, 128)],
                                  acc.at[ib.at[j, 1]], ssem.at[slot]).wait()

    def _weights(slot):
        ib, wb = idxq[slot], wq[slot % 2]
        for j in range(CROWS):
            @pl.loop(0, 128 // 16)
            def _g(g, j=j):
                s16 = ib[j, 0, pl.ds(g * 16, 16)]
                d16 = ib[j, 1, pl.ds(g * 16, 16)]
                z = (plsc.load_gather(svd_v, [s16, col0])
                     + plsc.load_gather(svd_v, [d16, col1]))
                z = jnp.where(z > 0, z, 0.2 * z)
                w = jnp.exp(z)
                wb[pl.ds(j * 128 + g * 16, 16)] = w
                plsc.addupdate_scatter(den_v, [d16], w)

    def _multiply(slot):
        rb, wb = rowsq[slot], wq[slot % 2]

        @pl.loop(0, CH, unroll=8)
        def _m(r):
            wv = plsc.load_gather(wb, [col0 + r])
            rb[r, pl.ds(0, 16)] = rb[r, pl.ds(0, 16)] * wv
            rb[r, pl.ds(16, 16)] = rb[r, pl.ds(16, 16)] * wv

    _idx_fire(0, 0)
    _idx_fire(1, 1)
    _idx_fire(2, 2)
    _idx_wait(0, 0)
    _gat_fire(0)
    _idx_wait(1, 1)
    _gat_fire(1)
    _weights(0)

    @pl.loop(0, NCHUNK // 4)
    def _k(k):
        for par in range(4):
            x = 4 * k + par
            _gat_wait(par)
            _multiply(par)
            _sc_fire(par)
            if par == 0:
                @pl.when(k >= 1)
                def _():
                    _sc_drain(3)
            else:
                _sc_drain(par - 1)
            _idx_wait(x + 2, (par + 2) % 4)
            _gat_fire((par + 2) % 4)
            _idx_fire(x + 3, (par + 3) % 4)
            if par == 3:
                @pl.when(k < NCHUNK // 4 - 1)
                def _():
                    _weights(0)
            else:
                _weights(par + 1)

    _gat_wait(NCHUNK % 4)
    _gat_wait((NCHUNK + 1) % 4)
    _sc_drain((NCHUNK - 1) % 4)
    _idx_wait(NCHUNK + 2, (NCHUNK + 2) % 4)

    pltpu.sync_copy(den_v, den_hbm.at[wid])
    plsc.subcore_barrier()

    @pl.when(sid == 0)
    def _():
        pltpu.sync_copy(acc, out_hbm.at[cid])


def _edge_call(hw, svd, edges, zeros, interpret=False):
    mesh = plsc.VectorSubcoreMesh(core_axis_name="c", subcore_axis_name="s",
                                  num_cores=NC, num_subcores=NS)
    cp = pltpu.CompilerParams(use_tc_tiling_on_sc=False)
    if "needs_layout_passes" in pltpu.CompilerParams.__dataclass_fields__:
        cp = dataclasses.replace(cp, needs_layout_passes=False)
    kern = pl.kernel(
        _edge_body,
        out_type=[jax.ShapeDtypeStruct((NC, NPAD, D), jnp.float32),
                  jax.ShapeDtypeStruct((NW, NPAD), jnp.float32)],
        mesh=mesh,
        scratch_types=[
            pltpu.VMEM((NPAD, 2), jnp.float32),
            pltpu.VMEM((NPAD,), jnp.float32),
            pltpu.VMEM((CROWS, 2, 128), jnp.int32),
            pltpu.VMEM((CROWS, 2, 128), jnp.int32),
            pltpu.VMEM((CROWS, 2, 128), jnp.int32),
            pltpu.VMEM((CROWS, 2, 128), jnp.int32),
            pltpu.VMEM((CH, D), jnp.float32),
            pltpu.VMEM((CH, D), jnp.float32),
            pltpu.VMEM((CH, D), jnp.float32),
            pltpu.VMEM((CH, D), jnp.float32),
            pltpu.VMEM((CH,), jnp.float32),
            pltpu.VMEM((CH,), jnp.float32),
            pltpu.VMEM_SHARED((NPAD, D), jnp.float32),
            pltpu.SemaphoreType.DMA((4,)),
            pltpu.SemaphoreType.DMA((4,)),
            pltpu.SemaphoreType.DMA((4,)),
        ],
        interpret=interpret,
        compiler_params=cp,
    )
    return kern(hw, svd, edges, zeros)


def _tc_call(body, out_shapes, args, grid=None, in_specs=None, out_specs=None,
             interpret=False):
    kwargs = {}
    if grid is not None:
        kwargs = dict(grid=grid, in_specs=in_specs, out_specs=out_specs)
    return pl.pallas_call(
        body,
        out_shape=out_shapes,
        interpret=interpret,
        **kwargs,
    )(*args)


def _full(shape):
    return pl.BlockSpec(shape, lambda i: tuple(0 for _ in shape))


def kernel(x, edge_index, batch, W_lin, b_lin, bn1_g, bn1_b, gat1_W, gat1_as,
           gat1_ad, gat1_b, bn2_g, bn2_b, gat2_W, gat2_as, gat2_ad, gat2_b,
           k1, conv1_w, lin1_W, k2, conv2_w, lin2_W):
    f32 = jnp.float32
    pad_idx = (jnp.arange(EPAD - E, dtype=jnp.int32) % (NPAD - N)) + N
    src2d = jnp.concatenate([edge_index[0], pad_idx]).reshape(EPAD // 128, 128)
    dst2d = jnp.concatenate([edge_index[1], pad_idx]).reshape(EPAD // 128, 128)
    edges = jnp.stack([src2d, dst2d], axis=1)
    edges = jnp.concatenate(
        [edges, jnp.full((8, 2, 128), N, jnp.int32)], axis=0)
    zeros = jnp.zeros((NPAD, D), f32)
    A1 = jnp.stack([gat1_as, gat1_ad], axis=1)
    A2 = jnp.stack([gat2_as, gat2_ad], axis=1)
    bl = b_lin.reshape(1, HID)
    g1 = bn1_g.reshape(1, HID); b1 = bn1_b.reshape(1, HID)
    g2 = bn2_g.reshape(1, HID); b2 = bn2_b.reshape(1, HID)
    gb1 = gat1_b.reshape(1, HID); gb2 = gat2_b.reshape(1, HID)
    kf = k1.reshape(NHEAD * NCLU, HID)
    cw = conv1_w.reshape(1, NHEAD)
    batch2d = batch.reshape(N, 1)

    h0 = _tc_call(
        _linproj_body,
        jax.ShapeDtypeStruct((N, HID), f32),
        (x, W_lin, bl),
        grid=(NBLK,),
        in_specs=[pl.BlockSpec((BLK, 128), lambda i: (i, 0)),
                  _full((128, HID)), _full((1, HID))],
        out_specs=pl.BlockSpec((BLK, HID), lambda i: (i, 0)))

    hw48_1, svd1 = _tc_call(
        _prep_body,
        [jax.ShapeDtypeStruct((NPAD, D), f32),
         jax.ShapeDtypeStruct((NPAD, 2), f32)],
        (h0, g1, b1, gat1_W, A1))
    acc1, den1 = _edge_call(hw48_1, svd1, edges, zeros)
    dtot1 = _tc_call(_denred_body, jax.ShapeDtypeStruct((NPAD, 1), f32),
                     (den1,))
    h1 = _tc_call(
        _combine_body,
        jax.ShapeDtypeStruct((N, HID), f32),
        (h0, hw48_1, svd1, acc1, dtot1, gb1),
        grid=(NBLK,),
        in_specs=[pl.BlockSpec((BLK, HID), lambda i: (i, 0)),
                  pl.BlockSpec((BLK, D), lambda i: (i, 0)),
                  pl.BlockSpec((BLK, 2), lambda i: (i, 0)),
                  pl.BlockSpec((NC, BLK, D), lambda i: (0, i, 0)),
                  pl.BlockSpec((BLK, 1), lambda i: (i, 0)),
                  _full((1, HID))],
        out_specs=pl.BlockSpec((BLK, HID), lambda i: (i, 0)))

    hw48_2, svd2 = _tc_call(
        _prep_body,
        [jax.ShapeDtypeStruct((NPAD, D), f32),
         jax.ShapeDtypeStruct((NPAD, 2), f32)],
        (h1, g2, b2, gat2_W, A2))
    acc2, den2 = _edge_call(hw48_2, svd2, edges, zeros)
    dtot2 = _tc_call(_denred_body, jax.ShapeDtypeStruct((NPAD, 1), f32),
                     (den2,))

    S, out_cs = _tc_call(
        _pool_body,
        [jax.ShapeDtypeStruct((N, NCLU), f32),
         jax.ShapeDtypeStruct((NG * NCLU, HID + 1), f32)],
        (h1, hw48_2, svd2, acc2, dtot2, gb2, batch2d, kf, cw),
        grid=(NBLK,),
        in_specs=[pl.BlockSpec((BLK, HID), lambda i: (i, 0)),
                  pl.BlockSpec((BLK, D), lambda i: (i, 0)),
                  pl.BlockSpec((BLK, 2), lambda i: (i, 0)),
                  pl.BlockSpec((NC, BLK, D), lambda i: (0, i, 0)),
                  pl.BlockSpec((BLK, 1), lambda i: (i, 0)),
                  _full((1, HID)),
                  pl.BlockSpec((BLK, 1), lambda i: (i, 0)),
                  _full((NHEAD * NCLU, HID)),
                  _full((1, NHEAD))],
        out_specs=[pl.BlockSpec((BLK, NCLU), lambda i: (i, 0)),
                   _full((NG * NCLU, HID + 1))])

    logp, kl = _tc_call(
        _head_body,
        [jax.ShapeDtypeStruct((NG, NCLU), f32),
         jax.ShapeDtypeStruct((1, 1), f32)],
        (S, out_cs, batch2d, lin1_W, lin2_W))
    return logp, kl[0, 0]


# speedup vs baseline: 48.2718x; 1.0262x over previous
import dataclasses
import functools

import jax
import jax.numpy as jnp
from jax import lax
from jax.experimental import pallas as pl
from jax.experimental.pallas import tpu as pltpu
from jax.experimental.pallas import tpu_sc as plsc

N = 10000
E = 320000
NPAD = 10016
EPAD = 327680
D = 32
HID = 32
NG = 16
NCLU = 10
NHEAD = 5
NC, NS = 2, 16
NW = NC * NS
EPW = EPAD // NW
CH = 128
CROWS = CH // 128
NCHUNK = EPW // CH
BLK = 1000
NBLK = N // BLK
EPS = 1e-15
_PREC = lax.Precision.HIGHEST


def _leaky(x, s):
    return jnp.where(x > 0, x, s * x)


def _dotT(a, b):
    return lax.dot_general(a, b, (((0,), (0,)), ((), ())),
                           preferred_element_type=jnp.float32,
                           precision=_PREC)


def _dot(a, b):
    return jnp.dot(a, b, preferred_element_type=jnp.float32, precision=_PREC)


def _pre_body(x_ref, wl_ref, bl_ref, g_ref, b_ref, w_ref, a_ref,
              h0_ref, hw_ref, svd_ref):
    h = _dot(x_ref[...], wl_ref[...]) + bl_ref[...]
    h0_ref[...] = h
    _prep_common(h, g_ref, b_ref, w_ref, a_ref, hw_ref, svd_ref)


def _prep_body(h_ref, g_ref, b_ref, w_ref, a_ref, hw_ref, svd_ref):
    _prep_common(h_ref[...], g_ref, b_ref, w_ref, a_ref, hw_ref, svd_ref)


def _prep_common(h, g_ref, b_ref, w_ref, a_ref, hw_ref, svd_ref):
    m = jnp.mean(h, axis=0)
    v = jnp.mean((h - m) ** 2, axis=0)
    hb = _leaky((h - m) / jnp.sqrt(v + 1e-5) * g_ref[...] + b_ref[...], 0.01)
    hw = _dot(hb, w_ref[...])
    svd = _dot(hw, a_ref[...])
    hw_ref[...] = jnp.concatenate(
        [hw, jnp.zeros((NPAD - N, D), jnp.float32)], axis=0)
    svd_ref[...] = jnp.concatenate(
        [svd, jnp.zeros((NPAD - N, 2), jnp.float32)], axis=0)


def _denred_body(den_ref, out_ref):
    out_ref[...] = _dotT(den_ref[...], jnp.ones((NW, 1), jnp.float32))


def _combine_vals(accv, dtot, hwv, svdv, biasv, hprev):
    accs = accv[0] + accv[1]
    z = svdv[:, 0:1] + svdv[:, 1:2]
    ws = jnp.exp(_leaky(z, 0.2))
    num = accs + ws * hwv
    den = dtot + ws
    return hprev + num / den + biasv


def _combine_body(h_ref, hw_ref, svd_ref, acc_ref, den_ref, bias_ref,
                  out_ref):
    out_ref[...] = _combine_vals(acc_ref[...], den_ref[...], hw_ref[...],
                                 svd_ref[...], bias_ref[...], h_ref[...])


def _pool_body(h_ref, hw_ref, svd_ref, acc_ref, den_ref, bias_ref,
               batch_ref, kf_ref, cw_ref, s_ref, cs_ref):
    h2 = _combine_vals(acc_ref[...], den_ref[...], hw_ref[...], svd_ref[...],
                       bias_ref[...], h_ref[...])
    kf = kf_ref[...]
    hn2 = jnp.sum(h2 * h2, axis=1, keepdims=True)
    kn2 = jnp.sum(kf * kf, axis=1)
    d2 = hn2 + kn2[None, :] - 2.0 * lax.dot_general(
        h2, kf, (((1,), (1,)), ((), ())),
        preferred_element_type=jnp.float32, precision=_PREC)
    d2 = jnp.maximum(d2, 0.0)
    dist = 1.0 / (1.0 + d2)
    i50h = lax.broadcasted_iota(jnp.int32, (NHEAD * NCLU, NHEAD), 0)
    i5h = lax.broadcasted_iota(jnp.int32, (NHEAD * NCLU, NHEAD), 1)
    M5 = (i50h // NCLU == i5h).astype(jnp.float32)
    dsum = _dot(dist, M5)
    dfull = lax.dot_general(dsum, M5, (((1,), (1,)), ((), ())),
                            preferred_element_type=jnp.float32,
                            precision=_PREC)
    distn = dist / dfull
    cwexp = lax.dot_general(cw_ref[...], M5, (((1,), (1,)), ((), ())),
                            preferred_element_type=jnp.float32,
                            precision=_PREC)
    i50k = lax.broadcasted_iota(jnp.int32, (NHEAD * NCLU, NCLU), 0)
    i10k = lax.broadcasted_iota(jnp.int32, (NHEAD * NCLU, NCLU), 1)
    M10 = (i50k % NCLU == i10k).astype(jnp.float32)
    S = _dot(distn * cwexp, M10)
    mx = jnp.max(S, axis=1, keepdims=True)
    e = jnp.exp(S - mx)
    S = e / jnp.sum(e, axis=1, keepdims=True)
    s_ref[...] = S
    bt = batch_ref[...]
    i16 = lax.broadcasted_iota(jnp.int32, (1, NG), 1)
    G = (bt == i16).astype(jnp.float32)
    iR0 = lax.broadcasted_iota(jnp.int32, (NG, NG * NCLU), 0)
    iR1 = lax.broadcasted_iota(jnp.int32, (NG, NG * NCLU), 1)
    R = (iR1 // NCLU == iR0).astype(jnp.float32)
    iT0 = lax.broadcasted_iota(jnp.int32, (NCLU, NG * NCLU), 0)
    iT1 = lax.broadcasted_iota(jnp.int32, (NCLU, NG * NCLU), 1)
    T = (iT1 % NCLU == iT0).astype(jnp.float32)
    P = _dot(G, R) * _dot(S, T)
    ext = jnp.concatenate([h2, jnp.ones((h2.shape[0], 1), jnp.float32)],
                          axis=1)

    @pl.when(pl.program_id(0) == 0)
    def _():
        cs_ref[...] = jnp.zeros_like(cs_ref)

    cs_ref[...] += _dotT(P, ext)


def _head_body(s_ref, cs_ref, batch_ref, lin1_ref, lin2_ref,
               logp_ref, kl_ref):
    out_cs = cs_ref[...]
    out1 = out_cs[:, :HID]
    colsum = out_cs[:, HID:HID + 1]
    iR0 = lax.broadcasted_iota(jnp.int32, (NG, NG * NCLU), 0)
    iR1 = lax.broadcasted_iota(jnp.int32, (NG, NG * NCLU), 1)
    R = (iR1 // NCLU == iR0).astype(jnp.float32)
    iT0 = lax.broadcasted_iota(jnp.int32, (NCLU, NG * NCLU), 0)
    iT1 = lax.broadcasted_iota(jnp.int32, (NCLU, NG * NCLU), 1)
    T = (iT1 % NCLU == iT0).astype(jnp.float32)
    x1 = _leaky(_dot(out1, lin1_ref[...]), 0.01)
    x2 = _dot(_dot(R, x1), lin2_ref[...])
    mx2 = jnp.max(x2, axis=1, keepdims=True)
    lse = jnp.log(jnp.sum(jnp.exp(x2 - mx2), axis=1, keepdims=True)) + mx2
    logp_ref[...] = x2 - lse
    cs16 = lax.dot_general(R * jnp.transpose(colsum), T,
                           (((1,), (1,)), ((), ())),
                           preferred_element_type=jnp.float32,
                           precision=_PREC)
    bt = batch_ref[...]
    i16 = lax.broadcasted_iota(jnp.int32, (1, NG), 1)
    G = (bt == i16).astype(jnp.float32)
    cs_pn = _dot(G, cs16)
    S = s_ref[...]
    Pn = (S * S) / jnp.maximum(cs_pn, EPS)
    Pn = Pn / jnp.sum(Pn, axis=1, keepdims=True)
    Pc = jnp.maximum(Pn, EPS)
    Sc = jnp.maximum(S, EPS)
    kl = jnp.sum(Pc * (jnp.log(Pc) - jnp.log(Sc))) / NG
    kl_ref[...] = jnp.reshape(kl, (1, 1))


def _edge_body(hw_hbm, svd_hbm, edge_hbm, zero_hbm, out_hbm, den_hbm,
               svd_v, den_v, i0, i1, i2, i3, r0, r1, r2, r3, w0, w1, acc,
               isem, gsem, ssem):
    cid = lax.axis_index("c")
    sid = lax.axis_index("s")
    wid = sid * NC + cid

    @pl.when(sid == 0)
    def _():
        pltpu.sync_copy(zero_hbm, acc)

    pltpu.sync_copy(svd_hbm, svd_v)
    z16 = jnp.zeros((16,), jnp.float32)

    @pl.loop(0, NPAD // 16)
    def _zero_den(i):
        den_v[pl.ds(i * 16, 16)] = z16

    plsc.subcore_barrier()

    col0 = lax.iota(jnp.int32, 16) * 0
    col1 = col0 + 1
    row0 = wid * (EPW // 128)
    idxq = (i0, i1, i2, i3)
    rowsq = (r0, r1, r2, r3)
    wq = (w0, w1)

    def _idx_fire(x, slot):
        pltpu.async_copy(edge_hbm.at[pl.ds(row0 + x * CROWS, CROWS)],
                         idxq[slot], isem.at[slot])

    def _idx_wait(x, slot):
        pltpu.make_async_copy(edge_hbm.at[pl.ds(row0 + x * CROWS, CROWS)],
                              idxq[slot], isem.at[slot]).wait()

    def _gat_fire(slot):
        ib, rb = idxq[slot], rowsq[slot]
        for j in range(CROWS):
            pltpu.async_copy(hw_hbm.at[ib.at[j, 0]],
                             rb.at[pl.ds(j * 128, 128)], gsem.at[slot])

    def _gat_wait(slot):
        ib, rb = idxq[slot], rowsq[slot]
        for j in range(CROWS):
            pltpu.make_async_copy(hw_hbm.at[ib.at[j, 0]],
                                  rb.at[pl.ds(j * 128, 128)],
                                  gsem.at[slot]).wait()

    def _sc_fire(slot):
        ib, rb = idxq[slot], rowsq[slot]
        for j in range(CROWS):
            pltpu.async_copy(rb.at[pl.ds(j * 128, 128)],
                             acc.at[ib.at[j, 1]], ssem.at[slot], add=True)

    def _sc_drain(slot):
        ib, rb = idxq[slot], rowsq[slot]
        for j in range(CROWS):
            pltpu.make_async_copy(rb.at[pl.ds(j * 128, 128)],
                                  acc.at[ib.at[j, 1]], ssem.at[slot]).wait()

    def _weights(slot):
        ib, wb = idxq[slot], wq[slot % 2]
        for j in range(CROWS):
            @pl.loop(0, 128 // 16)
            def _g(g, j=j):
                s16 = ib[j, 0, pl.ds(g * 16, 16)]
                d16 = ib[j, 1, pl.ds(g * 16, 16)]
                z = (plsc.load_gather(svd_v, [s16, col0])
                     + plsc.load_gather(svd_v, [d16, col1]))
                z = jnp.where(z > 0, z, 0.2 * z)
                w = jnp.exp(z)
                wb[pl.ds(j * 128 + g * 16, 16)] = w
                plsc.addupdate_scatter(den_v, [d16], w)

    def _multiply(slot):
        rb, wb = rowsq[slot], wq[slot % 2]

        @pl.loop(0, CH, unroll=8)
        def _m(r):
            wv = plsc.load_gather(wb, [col0 + r])
            rb[r, pl.ds(0, 16)] = rb[r, pl.ds(0, 16)] * wv
            rb[r, pl.ds(16, 16)] = rb[r, pl.ds(16, 16)] * wv

    _idx_fire(0, 0)
    _idx_fire(1, 1)
    _idx_fire(2, 2)
    _idx_wait(0, 0)
    _gat_fire(0)
    _idx_wait(1, 1)
    _gat_fire(1)
    _weights(0)

    @pl.loop(0, NCHUNK // 4)
    def _k(k):
        for par in range(4):
            x = 4 * k + par
            _gat_wait(par)
            _multiply(par)
            _sc_fire(par)
            if par == 0:
                @pl.when(k >= 1)
                def _():
                    _sc_drain(3)
            else:
                _sc_drain(par - 1)
            _idx_wait(x + 2, (par + 2) % 4)
            _gat_fire((par + 2) % 4)
            _idx_fire(x + 3, (par + 3) % 4)
            if par == 3:
                @pl.when(k < NCHUNK // 4 - 1)
                def _():
                    _weights(0)
            else:
                _weights(par + 1)

    _gat_wait(NCHUNK % 4)
    _gat_wait((NCHUNK + 1) % 4)
    _sc_drain((NCHUNK - 1) % 4)
    _idx_wait(NCHUNK + 2, (NCHUNK + 2) % 4)

    pltpu.sync_copy(den_v, den_hbm.at[wid])
    plsc.subcore_barrier()

    @pl.when(sid == 0)
    def _():
        pltpu.sync_copy(acc, out_hbm.at[cid])


def _edge_call(hw, svd, edges, zeros, interpret=False):
    mesh = plsc.VectorSubcoreMesh(core_axis_name="c", subcore_axis_name="s",
                                  num_cores=NC, num_subcores=NS)
    cp = pltpu.CompilerParams(use_tc_tiling_on_sc=False)
    if "needs_layout_passes" in pltpu.CompilerParams.__dataclass_fields__:
        cp = dataclasses.replace(cp, needs_layout_passes=False)
    kern = pl.kernel(
        _edge_body,
        out_type=[jax.ShapeDtypeStruct((NC, NPAD, D), jnp.float32),
                  jax.ShapeDtypeStruct((NW, NPAD), jnp.float32)],
        mesh=mesh,
        scratch_types=[
            pltpu.VMEM((NPAD, 2), jnp.float32),
            pltpu.VMEM((NPAD,), jnp.float32),
            pltpu.VMEM((CROWS, 2, 128), jnp.int32),
            pltpu.VMEM((CROWS, 2, 128), jnp.int32),
            pltpu.VMEM((CROWS, 2, 128), jnp.int32),
            pltpu.VMEM((CROWS, 2, 128), jnp.int32),
            pltpu.VMEM((CH, D), jnp.float32),
            pltpu.VMEM((CH, D), jnp.float32),
            pltpu.VMEM((CH, D), jnp.float32),
            pltpu.VMEM((CH, D), jnp.float32),
            pltpu.VMEM((CH,), jnp.float32),
            pltpu.VMEM((CH,), jnp.float32),
            pltpu.VMEM_SHARED((NPAD, D), jnp.float32),
            pltpu.SemaphoreType.DMA((4,)),
            pltpu.SemaphoreType.DMA((4,)),
            pltpu.SemaphoreType.DMA((4,)),
        ],
        interpret=interpret,
        compiler_params=cp,
    )
    return kern(hw, svd, edges, zeros)


def _tc_call(body, out_shapes, args, grid=None, in_specs=None, out_specs=None,
             interpret=False):
    kwargs = {}
    if grid is not None:
        kwargs = dict(grid=grid, in_specs=in_specs, out_specs=out_specs)
    return pl.pallas_call(
        body,
        out_shape=out_shapes,
        interpret=interpret,
        **kwargs,
    )(*args)


def _full(shape):
    return pl.BlockSpec(shape, lambda i: tuple(0 for _ in shape))


def kernel(x, edge_index, batch, W_lin, b_lin, bn1_g, bn1_b, gat1_W, gat1_as,
           gat1_ad, gat1_b, bn2_g, bn2_b, gat2_W, gat2_as, gat2_ad, gat2_b,
           k1, conv1_w, lin1_W, k2, conv2_w, lin2_W):
    f32 = jnp.float32
    pad_idx = (jnp.arange(EPAD - E, dtype=jnp.int32) % (NPAD - N)) + N
    src2d = jnp.concatenate([edge_index[0], pad_idx]).reshape(EPAD // 128, 128)
    dst2d = jnp.concatenate([edge_index[1], pad_idx]).reshape(EPAD // 128, 128)
    edges = jnp.stack([src2d, dst2d], axis=1)
    edges = jnp.concatenate(
        [edges, jnp.full((8, 2, 128), N, jnp.int32)], axis=0)
    zeros = jnp.zeros((NPAD, D), f32)
    A1 = jnp.stack([gat1_as, gat1_ad], axis=1)
    A2 = jnp.stack([gat2_as, gat2_ad], axis=1)
    bl = b_lin.reshape(1, HID)
    g1 = bn1_g.reshape(1, HID); b1 = bn1_b.reshape(1, HID)
    g2 = bn2_g.reshape(1, HID); b2 = bn2_b.reshape(1, HID)
    gb1 = gat1_b.reshape(1, HID); gb2 = gat2_b.reshape(1, HID)
    kf = k1.reshape(NHEAD * NCLU, HID)
    cw = conv1_w.reshape(1, NHEAD)
    batch2d = batch.reshape(N, 1)

    h0, hw48_1, svd1 = _tc_call(
        _pre_body,
        [jax.ShapeDtypeStruct((N, HID), f32),
         jax.ShapeDtypeStruct((NPAD, D), f32),
         jax.ShapeDtypeStruct((NPAD, 2), f32)],
        (x, W_lin, bl, g1, b1, gat1_W, A1))
    acc1, den1 = _edge_call(hw48_1, svd1, edges, zeros)
    dtot1 = _tc_call(_denred_body, jax.ShapeDtypeStruct((NPAD, 1), f32),
                     (den1,))
    h1 = _tc_call(
        _combine_body,
        jax.ShapeDtypeStruct((N, HID), f32),
        (h0, hw48_1, svd1, acc1, dtot1, gb1),
        grid=(NBLK,),
        in_specs=[pl.BlockSpec((BLK, HID), lambda i: (i, 0)),
                  pl.BlockSpec((BLK, D), lambda i: (i, 0)),
                  pl.BlockSpec((BLK, 2), lambda i: (i, 0)),
                  pl.BlockSpec((NC, BLK, D), lambda i: (0, i, 0)),
                  pl.BlockSpec((BLK, 1), lambda i: (i, 0)),
                  _full((1, HID))],
        out_specs=pl.BlockSpec((BLK, HID), lambda i: (i, 0)))

    hw48_2, svd2 = _tc_call(
        _prep_body,
        [jax.ShapeDtypeStruct((NPAD, D), f32),
         jax.ShapeDtypeStruct((NPAD, 2), f32)],
        (h1, g2, b2, gat2_W, A2))
    acc2, den2 = _edge_call(hw48_2, svd2, edges, zeros)
    dtot2 = _tc_call(_denred_body, jax.ShapeDtypeStruct((NPAD, 1), f32),
                     (den2,))

    S, out_cs = _tc_call(
        _pool_body,
        [jax.ShapeDtypeStruct((N, NCLU), f32),
         jax.ShapeDtypeStruct((NG * NCLU, HID + 1), f32)],
        (h1, hw48_2, svd2, acc2, dtot2, gb2, batch2d, kf, cw),
        grid=(NBLK,),
        in_specs=[pl.BlockSpec((BLK, HID), lambda i: (i, 0)),
                  pl.BlockSpec((BLK, D), lambda i: (i, 0)),
                  pl.BlockSpec((BLK, 2), lambda i: (i, 0)),
                  pl.BlockSpec((NC, BLK, D), lambda i: (0, i, 0)),
                  pl.BlockSpec((BLK, 1), lambda i: (i, 0)),
                  _full((1, HID)),
                  pl.BlockSpec((BLK, 1), lambda i: (i, 0)),
                  _full((NHEAD * NCLU, HID)),
                  _full((1, NHEAD))],
        out_specs=[pl.BlockSpec((BLK, NCLU), lambda i: (i, 0)),
                   _full((NG * NCLU, HID + 1))])

    logp, kl = _tc_call(
        _head_body,
        [jax.ShapeDtypeStruct((NG, NCLU), f32),
         jax.ShapeDtypeStruct((1, 1), f32)],
        (S, out_cs, batch2d, lin1_W, lin2_W))
    return logp, kl[0, 0]


# speedup vs baseline: 51.9121x; 1.0754x over previous
import dataclasses
import functools

import jax
import jax.numpy as jnp
from jax import lax
from jax.experimental import pallas as pl
from jax.experimental.pallas import tpu as pltpu
from jax.experimental.pallas import tpu_sc as plsc

N = 10000
E = 320000
NPAD = 10016
EPAD = 327680
D = 32
HID = 32
NG = 16
NCLU = 10
NHEAD = 5
NC, NS = 2, 16
NW = NC * NS
EPW = EPAD // NW
CH = 128
CROWS = CH // 128
NCHUNK = EPW // CH
BLK = 2000
NBLK = N // BLK
EPS = 1e-15
_PREC = lax.Precision.HIGHEST


def _leaky(x, s):
    return jnp.where(x > 0, x, s * x)


def _dotT(a, b):
    return lax.dot_general(a, b, (((0,), (0,)), ((), ())),
                           preferred_element_type=jnp.float32,
                           precision=_PREC)


def _dot(a, b):
    return jnp.dot(a, b, preferred_element_type=jnp.float32, precision=_PREC)


def _pre_body(x_ref, wl_ref, bl_ref, g_ref, b_ref, w_ref, a_ref,
              h0_ref, hw_ref, svd_ref):
    h = _dot(x_ref[...], wl_ref[...]) + bl_ref[...]
    h0_ref[...] = h
    _prep_common(h, g_ref, b_ref, w_ref, a_ref, hw_ref, svd_ref)


def _prep_body(h_ref, g_ref, b_ref, w_ref, a_ref, hw_ref, svd_ref):
    _prep_common(h_ref[...], g_ref, b_ref, w_ref, a_ref, hw_ref, svd_ref)


def _prep_common(h, g_ref, b_ref, w_ref, a_ref, hw_ref, svd_ref):
    m = jnp.mean(h, axis=0)
    v = jnp.mean((h - m) ** 2, axis=0)
    hb = _leaky((h - m) / jnp.sqrt(v + 1e-5) * g_ref[...] + b_ref[...], 0.01)
    hw = _dot(hb, w_ref[...])
    svd = _dot(hw, a_ref[...])
    hw_ref[...] = jnp.concatenate(
        [hw, jnp.zeros((NPAD - N, D), jnp.float32)], axis=0)
    svd_ref[...] = jnp.concatenate(
        [svd, jnp.zeros((NPAD - N, 2), jnp.float32)], axis=0)


def _denred_body(den_ref, out_ref):
    out_ref[...] = _dotT(den_ref[...], jnp.ones((NW, 1), jnp.float32))


def _combine_vals(accv, dtot, hwv, svdv, biasv, hprev):
    accs = accv[0] + accv[1]
    z = svdv[:, 0:1] + svdv[:, 1:2]
    ws = jnp.exp(_leaky(z, 0.2))
    num = accs + ws * hwv
    den = dtot + ws
    return hprev + num / den + biasv


def _combine_body(h_ref, hw_ref, svd_ref, acc_ref, den_ref, bias_ref,
                  out_ref):
    out_ref[...] = _combine_vals(acc_ref[...], den_ref[...], hw_ref[...],
                                 svd_ref[...], bias_ref[...], h_ref[...])


def _pool_body(h_ref, hw_ref, svd_ref, acc_ref, den_ref, bias_ref,
               batch_ref, kf_ref, cw_ref, s_ref, cs_ref):
    h2 = _combine_vals(acc_ref[...], den_ref[...], hw_ref[...], svd_ref[...],
                       bias_ref[...], h_ref[...])
    kf = kf_ref[...]
    hn2 = jnp.sum(h2 * h2, axis=1, keepdims=True)
    kn2 = jnp.sum(kf * kf, axis=1)
    d2 = hn2 + kn2[None, :] - 2.0 * lax.dot_general(
        h2, kf, (((1,), (1,)), ((), ())),
        preferred_element_type=jnp.float32, precision=_PREC)
    d2 = jnp.maximum(d2, 0.0)
    dist = 1.0 / (1.0 + d2)
    i50h = lax.broadcasted_iota(jnp.int32, (NHEAD * NCLU, NHEAD), 0)
    i5h = lax.broadcasted_iota(jnp.int32, (NHEAD * NCLU, NHEAD), 1)
    M5 = (i50h // NCLU == i5h).astype(jnp.float32)
    dsum = _dot(dist, M5)
    dfull = lax.dot_general(dsum, M5, (((1,), (1,)), ((), ())),
                            preferred_element_type=jnp.float32,
                            precision=_PREC)
    distn = dist / dfull
    cwexp = lax.dot_general(cw_ref[...], M5, (((1,), (1,)), ((), ())),
                            preferred_element_type=jnp.float32,
                            precision=_PREC)
    i50k = lax.broadcasted_iota(jnp.int32, (NHEAD * NCLU, NCLU), 0)
    i10k = lax.broadcasted_iota(jnp.int32, (NHEAD * NCLU, NCLU), 1)
    M10 = (i50k % NCLU == i10k).astype(jnp.float32)
    S = _dot(distn * cwexp, M10)
    mx = jnp.max(S, axis=1, keepdims=True)
    e = jnp.exp(S - mx)
    S = e / jnp.sum(e, axis=1, keepdims=True)
    s_ref[...] = S
    bt = batch_ref[...]
    i16 = lax.broadcasted_iota(jnp.int32, (1, NG), 1)
    G = (bt == i16).astype(jnp.float32)
    iR0 = lax.broadcasted_iota(jnp.int32, (NG, NG * NCLU), 0)
    iR1 = lax.broadcasted_iota(jnp.int32, (NG, NG * NCLU), 1)
    R = (iR1 // NCLU == iR0).astype(jnp.float32)
    iT0 = lax.broadcasted_iota(jnp.int32, (NCLU, NG * NCLU), 0)
    iT1 = lax.broadcasted_iota(jnp.int32, (NCLU, NG * NCLU), 1)
    T = (iT1 % NCLU == iT0).astype(jnp.float32)
    P = _dot(G, R) * _dot(S, T)
    ext = jnp.concatenate([h2, jnp.ones((h2.shape[0], 1), jnp.float32)],
                          axis=1)

    @pl.when(pl.program_id(0) == 0)
    def _():
        cs_ref[...] = jnp.zeros_like(cs_ref)

    cs_ref[...] += _dotT(P, ext)


def _head_body(s_ref, cs_ref, batch_ref, lin1_ref, lin2_ref,
               logp_ref, kl_ref):
    out_cs = cs_ref[...]
    out1 = out_cs[:, :HID]
    colsum = out_cs[:, HID:HID + 1]
    iR0 = lax.broadcasted_iota(jnp.int32, (NG, NG * NCLU), 0)
    iR1 = lax.broadcasted_iota(jnp.int32, (NG, NG * NCLU), 1)
    R = (iR1 // NCLU == iR0).astype(jnp.float32)
    iT0 = lax.broadcasted_iota(jnp.int32, (NCLU, NG * NCLU), 0)
    iT1 = lax.broadcasted_iota(jnp.int32, (NCLU, NG * NCLU), 1)
    T = (iT1 % NCLU == iT0).astype(jnp.float32)
    x1 = _leaky(_dot(out1, lin1_ref[...]), 0.01)
    x2 = _dot(_dot(R, x1), lin2_ref[...])
    mx2 = jnp.max(x2, axis=1, keepdims=True)
    lse = jnp.log(jnp.sum(jnp.exp(x2 - mx2), axis=1, keepdims=True)) + mx2
    logp_ref[...] = x2 - lse
    cs16 = lax.dot_general(R * jnp.transpose(colsum), T,
                           (((1,), (1,)), ((), ())),
                           preferred_element_type=jnp.float32,
                           precision=_PREC)
    bt = batch_ref[...]
    i16 = lax.broadcasted_iota(jnp.int32, (1, NG), 1)
    G = (bt == i16).astype(jnp.float32)
    cs_pn = _dot(G, cs16)
    S = s_ref[...]
    Pn = (S * S) / jnp.maximum(cs_pn, EPS)
    Pn = Pn / jnp.sum(Pn, axis=1, keepdims=True)
    Pc = jnp.maximum(Pn, EPS)
    Sc = jnp.maximum(S, EPS)
    kl = jnp.sum(Pc * (jnp.log(Pc) - jnp.log(Sc))) / NG
    kl_ref[...] = jnp.reshape(kl, (1, 1))


def _edge_body(hw_hbm, svd_hbm, edge_hbm, zero_hbm, out_hbm, den_hbm,
               svd_v, den_v, i0, i1, i2, i3, r0, r1, r2, r3, w0, w1, acc,
               isem, gsem, ssem):
    cid = lax.axis_index("c")
    sid = lax.axis_index("s")
    wid = sid * NC + cid

    @pl.when(sid == 0)
    def _():
        pltpu.sync_copy(zero_hbm, acc)

    pltpu.sync_copy(svd_hbm, svd_v)
    z16 = jnp.zeros((16,), jnp.float32)

    @pl.loop(0, NPAD // 16)
    def _zero_den(i):
        den_v[pl.ds(i * 16, 16)] = z16

    plsc.subcore_barrier()

    col0 = lax.iota(jnp.int32, 16) * 0
    col1 = col0 + 1
    row0 = wid * (EPW // 128)
    idxq = (i0, i1, i2, i3)
    rowsq = (r0, r1, r2, r3)
    wq = (w0, w1)

    def _idx_fire(x, slot):
        pltpu.async_copy(edge_hbm.at[pl.ds(row0 + x * CROWS, CROWS)],
                         idxq[slot], isem.at[slot])

    def _idx_wait(x, slot):
        pltpu.make_async_copy(edge_hbm.at[pl.ds(row0 + x * CROWS, CROWS)],
                              idxq[slot], isem.at[slot]).wait()

    def _gat_fire(slot):
        ib, rb = idxq[slot], rowsq[slot]
        for j in range(CROWS):
            pltpu.async_copy(hw_hbm.at[ib.at[j, 0]],
                             rb.at[pl.ds(j * 128, 128)], gsem.at[slot])

    def _gat_wait(slot):
        ib, rb = idxq[slot], rowsq[slot]
        for j in range(CROWS):
            pltpu.make_async_copy(hw_hbm.at[ib.at[j, 0]],
                                  rb.at[pl.ds(j * 128, 128)],
                                  gsem.at[slot]).wait()

    def _sc_fire(slot):
        ib, rb = idxq[slot], rowsq[slot]
        for j in range(CROWS):
            pltpu.async_copy(rb.at[pl.ds(j * 128, 128)],
                             acc.at[ib.at[j, 1]], ssem.at[slot], add=True)

    def _sc_drain(slot):
        ib, rb = idxq[slot], rowsq[slot]
        for j in range(CROWS):
            pltpu.make_async_copy(rb.at[pl.ds(j * 128, 128)],
                                  acc.at[ib.at[j, 1]], ssem.at[slot]).wait()

    def _weights(slot):
        ib, wb = idxq[slot], wq[slot % 2]
        for j in range(CROWS):
            @pl.loop(0, 128 // 16)
            def _g(g, j=j):
                s16 = ib[j, 0, pl.ds(g * 16, 16)]
                d16 = ib[j, 1, pl.ds(g * 16, 16)]
                z = (plsc.load_gather(svd_v, [s16, col0])
                     + plsc.load_gather(svd_v, [d16, col1]))
                z = jnp.where(z > 0, z, 0.2 * z)
                w = jnp.exp(z)
                wb[pl.ds(j * 128 + g * 16, 16)] = w
                plsc.addupdate_scatter(den_v, [d16], w)

    def _multiply(slot):
        rb, wb = rowsq[slot], wq[slot % 2]

        @pl.loop(0, CH, unroll=8)
        def _m(r):
            wv = plsc.load_gather(wb, [col0 + r])
            rb[r, pl.ds(0, 16)] = rb[r, pl.ds(0, 16)] * wv
            rb[r, pl.ds(16, 16)] = rb[r, pl.ds(16, 16)] * wv

    _idx_fire(0, 0)
    _idx_fire(1, 1)
    _idx_fire(2, 2)
    _idx_wait(0, 0)
    _gat_fire(0)
    _idx_wait(1, 1)
    _gat_fire(1)
    _weights(0)

    @pl.loop(0, NCHUNK // 4)
    def _k(k):
        for par in range(4):
            x = 4 * k + par
            _gat_wait(par)
            _multiply(par)
            _sc_fire(par)
            if par == 0:
                @pl.when(k >= 1)
                def _():
                    _sc_drain(3)
            else:
                _sc_drain(par - 1)
            _idx_wait(x + 2, (par + 2) % 4)
            _gat_fire((par + 2) % 4)
            _idx_fire(x + 3, (par + 3) % 4)
            if par == 3:
                @pl.when(k < NCHUNK // 4 - 1)
                def _():
                    _weights(0)
            else:
                _weights(par + 1)

    _gat_wait(NCHUNK % 4)
    _gat_wait((NCHUNK + 1) % 4)
    _sc_drain((NCHUNK - 1) % 4)
    _idx_wait(NCHUNK + 2, (NCHUNK + 2) % 4)

    pltpu.sync_copy(den_v, den_hbm.at[wid])
    plsc.subcore_barrier()

    @pl.when(sid == 0)
    def _():
        pltpu.sync_copy(acc, out_hbm.at[cid])


def _edge_call(hw, svd, edges, zeros, interpret=False):
    mesh = plsc.VectorSubcoreMesh(core_axis_name="c", subcore_axis_name="s",
                                  num_cores=NC, num_subcores=NS)
    cp = pltpu.CompilerParams(use_tc_tiling_on_sc=False)
    if "needs_layout_passes" in pltpu.CompilerParams.__dataclass_fields__:
        cp = dataclasses.replace(cp, needs_layout_passes=False)
    kern = pl.kernel(
        _edge_body,
        out_type=[jax.ShapeDtypeStruct((NC, NPAD, D), jnp.float32),
                  jax.ShapeDtypeStruct((NW, NPAD), jnp.float32)],
        mesh=mesh,
        scratch_types=[
            pltpu.VMEM((NPAD, 2), jnp.float32),
            pltpu.VMEM((NPAD,), jnp.float32),
            pltpu.VMEM((CROWS, 2, 128), jnp.int32),
            pltpu.VMEM((CROWS, 2, 128), jnp.int32),
            pltpu.VMEM((CROWS, 2, 128), jnp.int32),
            pltpu.VMEM((CROWS, 2, 128), jnp.int32),
            pltpu.VMEM((CH, D), jnp.float32),
            pltpu.VMEM((CH, D), jnp.float32),
            pltpu.VMEM((CH, D), jnp.float32),
            pltpu.VMEM((CH, D), jnp.float32),
            pltpu.VMEM((CH,), jnp.float32),
            pltpu.VMEM((CH,), jnp.float32),
            pltpu.VMEM_SHARED((NPAD, D), jnp.float32),
            pltpu.SemaphoreType.DMA((4,)),
            pltpu.SemaphoreType.DMA((4,)),
            pltpu.SemaphoreType.DMA((4,)),
        ],
        interpret=interpret,
        compiler_params=cp,
    )
    return kern(hw, svd, edges, zeros)


def _tc_call(body, out_shapes, args, grid=None, in_specs=None, out_specs=None,
             interpret=False):
    kwargs = {}
    if grid is not None:
        kwargs = dict(grid=grid, in_specs=in_specs, out_specs=out_specs)
    return pl.pallas_call(
        body,
        out_shape=out_shapes,
        interpret=interpret,
        **kwargs,
    )(*args)


def _full(shape):
    return pl.BlockSpec(shape, lambda i: tuple(0 for _ in shape))


def kernel(x, edge_index, batch, W_lin, b_lin, bn1_g, bn1_b, gat1_W, gat1_as,
           gat1_ad, gat1_b, bn2_g, bn2_b, gat2_W, gat2_as, gat2_ad, gat2_b,
           k1, conv1_w, lin1_W, k2, conv2_w, lin2_W):
    f32 = jnp.float32
    pad_idx = (jnp.arange(EPAD - E, dtype=jnp.int32) % (NPAD - N)) + N
    src2d = jnp.concatenate([edge_index[0], pad_idx]).reshape(EPAD // 128, 128)
    dst2d = jnp.concatenate([edge_index[1], pad_idx]).reshape(EPAD // 128, 128)
    edges = jnp.stack([src2d, dst2d], axis=1)
    edges = jnp.concatenate(
        [edges, jnp.full((8, 2, 128), N, jnp.int32)], axis=0)
    zeros = jnp.zeros((NPAD, D), f32)
    A1 = jnp.stack([gat1_as, gat1_ad], axis=1)
    A2 = jnp.stack([gat2_as, gat2_ad], axis=1)
    bl = b_lin.reshape(1, HID)
    g1 = bn1_g.reshape(1, HID); b1 = bn1_b.reshape(1, HID)
    g2 = bn2_g.reshape(1, HID); b2 = bn2_b.reshape(1, HID)
    gb1 = gat1_b.reshape(1, HID); gb2 = gat2_b.reshape(1, HID)
    kf = k1.reshape(NHEAD * NCLU, HID)
    cw = conv1_w.reshape(1, NHEAD)
    batch2d = batch.reshape(N, 1)

    h0, hw48_1, svd1 = _tc_call(
        _pre_body,
        [jax.ShapeDtypeStruct((N, HID), f32),
         jax.ShapeDtypeStruct((NPAD, D), f32),
         jax.ShapeDtypeStruct((NPAD, 2), f32)],
        (x, W_lin, bl, g1, b1, gat1_W, A1))
    acc1, den1 = _edge_call(hw48_1, svd1, edges, zeros)
    dtot1 = _tc_call(_denred_body, jax.ShapeDtypeStruct((NPAD, 1), f32),
                     (den1,))
    h1 = _tc_call(
        _combine_body,
        jax.ShapeDtypeStruct((N, HID), f32),
        (h0, hw48_1, svd1, acc1, dtot1, gb1),
        grid=(NBLK,),
        in_specs=[pl.BlockSpec((BLK, HID), lambda i: (i, 0)),
                  pl.BlockSpec((BLK, D), lambda i: (i, 0)),
                  pl.BlockSpec((BLK, 2), lambda i: (i, 0)),
                  pl.BlockSpec((NC, BLK, D), lambda i: (0, i, 0)),
                  pl.BlockSpec((BLK, 1), lambda i: (i, 0)),
                  _full((1, HID))],
        out_specs=pl.BlockSpec((BLK, HID), lambda i: (i, 0)))

    hw48_2, svd2 = _tc_call(
        _prep_body,
        [jax.ShapeDtypeStruct((NPAD, D), f32),
         jax.ShapeDtypeStruct((NPAD, 2), f32)],
        (h1, g2, b2, gat2_W, A2))
    acc2, den2 = _edge_call(hw48_2, svd2, edges, zeros)
    dtot2 = _tc_call(_denred_body, jax.ShapeDtypeStruct((NPAD, 1), f32),
                     (den2,))

    S, out_cs = _tc_call(
        _pool_body,
        [jax.ShapeDtypeStruct((N, NCLU), f32),
         jax.ShapeDtypeStruct((NG * NCLU, HID + 1), f32)],
        (h1, hw48_2, svd2, acc2, dtot2, gb2, batch2d, kf, cw),
        grid=(NBLK,),
        in_specs=[pl.BlockSpec((BLK, HID), lambda i: (i, 0)),
                  pl.BlockSpec((BLK, D), lambda i: (i, 0)),
                  pl.BlockSpec((BLK, 2), lambda i: (i, 0)),
                  pl.BlockSpec((NC, BLK, D), lambda i: (0, i, 0)),
                  pl.BlockSpec((BLK, 1), lambda i: (i, 0)),
                  _full((1, HID)),
                  pl.BlockSpec((BLK, 1), lambda i: (i, 0)),
                  _full((NHEAD * NCLU, HID)),
                  _full((1, NHEAD))],
        out_specs=[pl.BlockSpec((BLK, NCLU), lambda i: (i, 0)),
                   _full((NG * NCLU, HID + 1))])

    logp, kl = _tc_call(
        _head_body,
        [jax.ShapeDtypeStruct((NG, NCLU), f32),
         jax.ShapeDtypeStruct((1, 1), f32)],
        (S, out_cs, batch2d, lin1_W, lin2_W))
    return logp, kl[0, 0]
